# Initial kernel scaffold; baseline (speedup 1.0000x reference)
import functools

import jax
import jax.numpy as jnp
from jax import lax
from jax.experimental import pallas as pl
from jax.experimental.pallas import tpu as pltpu
from jax.experimental.pallas import tpu_sc as plsc

F32 = jnp.float32
I32 = jnp.int32

N = 10000
NP = 10240
E = 160000
EP = 163840
EMB = 256
NBLK = 1024
NGRID = NP // NBLK
ECH = 128
EROWS = EP // ECH
TPR = EROWS // 16
TPR2 = EROWS // 32

_MESH = plsc.VectorSubcoreMesh(core_axis_name="c", subcore_axis_name="s")


def _zero16():
    return jnp.zeros((16,), F32)


def _rsqrt16(v):
    i = plsc.bitcast(v, I32)
    i = jnp.int32(0x5F3759DF) - lax.shift_right_logical(i, 1)
    y = plsc.bitcast(i, F32)
    for _ in range(3):
        y = y * (1.5 - 0.5 * v * y * y)
    return y


@functools.partial(
    pl.kernel,
    mesh=_MESH,
    out_type=[
        jax.ShapeDtypeStruct((NP,), F32),
        jax.ShapeDtypeStruct((NP,), F32),
        jax.ShapeDtypeStruct((NP,), F32),
        jax.ShapeDtypeStruct((2 * NP * 48,), F32),
    ],
    scratch_types=[
        pltpu.VMEM_SHARED((NP,), F32),
        pltpu.VMEM_SHARED((NP * 48,), F32),
        pltpu.VMEM((3840,), F32),
        pltpu.VMEM((128,), F32),
        pltpu.VMEM((TPR, ECH), I32),
        pltpu.VMEM((NP,), F32),
        pltpu.VMEM((NP,), F32),
        pltpu.VMEM((NP,), F32),
        pltpu.VMEM((NP,), F32),
        pltpu.VMEM((TPR2, ECH), I32),
        pltpu.VMEM((TPR2, ECH), I32),
        pltpu.VMEM((TPR2, ECH), I32),
        pltpu.VMEM((TPR2, ECH), I32),
        pltpu.VMEM((TPR2, ECH), I32),
        pltpu.VMEM((128,), F32),
        pltpu.VMEM((8, ECH), I32),
    ],
)
def _sc_prep(row2d, col2d, ea0, ea1, ea2,
             dis_o, sdeg_o, dinv_o, t_o,
             deg_sp, t_sp, zb, ones, rowv, degv, disv, sdegv, dinvv,
             rowt, colt, e0v, e1v, e2v, valb, idxb):
    c = lax.axis_index("c")
    s = lax.axis_index("s")

    def _fill(i, _):
        zb[pl.ds(i * 16, 16)] = _zero16()
        return 0
    lax.fori_loop(0, 240, _fill, 0)

    def _fill1(i, _):
        ones[pl.ds(i * 16, 16)] = jnp.ones((16,), F32)
        return 0
    lax.fori_loop(0, 8, _fill1, 0)

    pltpu.sync_copy(zb.at[pl.ds(0, 640)], deg_sp.at[pl.ds(s * 640, 640)])

    def _zt(i, _):
        pltpu.sync_copy(zb, t_sp.at[pl.ds(s * 30720 + i * 3840, 3840)])
        return 0
    lax.fori_loop(0, 8, _zt, 0)
    plsc.subcore_barrier()

    pltpu.sync_copy(row2d.at[pl.ds(s * TPR, TPR)], rowv)

    def _deg(j, _):
        pltpu.sync_copy(ones, deg_sp.at[rowv.at[j]], add=True)
        return 0
    lax.fori_loop(0, TPR, _deg, 0)
    plsc.subcore_barrier()

    pltpu.sync_copy(deg_sp, degv)

    def _dis(k, _):
        v = degv[pl.ds(k * 16, 16)] + 1.0
        y = _rsqrt16(v)
        disv[pl.ds(k * 16, 16)] = y
        sdegv[pl.ds(k * 16, 16)] = v * y
        dinvv[pl.ds(k * 16, 16)] = y * y
        return 0
    lax.fori_loop(0, NP // 16, _dis, 0)

    @pl.when(c == 0)
    def _():
        pltpu.sync_copy(disv.at[pl.ds(s * 640, 640)], dis_o.at[pl.ds(s * 640, 640)])
        pltpu.sync_copy(sdegv.at[pl.ds(s * 640, 640)], sdeg_o.at[pl.ds(s * 640, 640)])
        pltpu.sync_copy(dinvv.at[pl.ds(s * 640, 640)], dinv_o.at[pl.ds(s * 640, 640)])

    base = c * (EROWS // 2) + s * TPR2
    pltpu.sync_copy(row2d.at[pl.ds(base, TPR2)], rowt)
    pltpu.sync_copy(col2d.at[pl.ds(base, TPR2)], colt)
    pltpu.sync_copy(ea0.at[pl.ds(base, TPR2)], e0v)
    pltpu.sync_copy(ea1.at[pl.ds(base, TPR2)], e1v)
    pltpu.sync_copy(ea2.at[pl.ds(base, TPR2)], e2v)

    def _tchunk(j, _):
        def _lane(k, _):
            sl = pl.ds(k * 16, 16)
            ridx = rowt.at[j][sl]
            valb[sl] = plsc.load_gather(disv, [ridx])
            cv = colt.at[j][sl] * 48
            idxb[0, sl] = cv + e0v.at[j][sl]
            idxb[1, sl] = cv + 16 + e1v.at[j][sl]
            idxb[2, sl] = cv + 32 + e2v.at[j][sl]
            return 0
        lax.fori_loop(0, 8, _lane, 0)
        pltpu.sync_copy(valb, t_sp.at[idxb.at[0]], add=True)
        pltpu.sync_copy(valb, t_sp.at[idxb.at[1]], add=True)
        pltpu.sync_copy(valb, t_sp.at[idxb.at[2]], add=True)
        return 0
    lax.fori_loop(0, TPR2, _tchunk, 0)
    plsc.subcore_barrier()

    pltpu.sync_copy(t_sp.at[pl.ds(s * 30720, 30720)],
                    t_o.at[pl.ds(c * (NP * 48) + s * 30720, 30720)])


@functools.partial(
    pl.kernel,
    mesh=_MESH,
    out_type=jax.ShapeDtypeStruct((2 * NP, 128), F32),
    scratch_types=[
        pltpu.VMEM_SHARED((NP, 128), F32),
        pltpu.VMEM((TPR, ECH), I32),
        pltpu.VMEM((TPR, ECH), I32),
        pltpu.VMEM((ECH, 128), F32),
        pltpu.VMEM((64, 128), F32),
        pltpu.SemaphoreType.DMA,
    ],
)
def _sc_spmm(hs2, row2d, col2d, g_o, acc, rowv, colv, dbuf, zbuf, sem):
    c = lax.axis_index("c")
    s = lax.axis_index("s")

    def _zrow(r, _):
        rr = zbuf.at[r]

        def _zl(k, _):
            rr[pl.ds(k * 16, 16)] = _zero16()
            return 0
        lax.fori_loop(0, 8, _zl, 0)
        return 0
    lax.fori_loop(0, 64, _zrow, 0)

    def _zacc(i, _):
        pltpu.sync_copy(zbuf, acc.at[pl.ds(s * 640 + i * 64, 64)])
        return 0
    lax.fori_loop(0, 10, _zacc, 0)

    pltpu.sync_copy(row2d.at[pl.ds(s * TPR, TPR)], rowv)
    pltpu.sync_copy(col2d.at[pl.ds(s * TPR, TPR)], colv)

    off = c * NP

    def _addoff(j, _):
        rr = rowv.at[j]

        def _al(k, _):
            sl = pl.ds(k * 16, 16)
            rr[sl] = rr[sl] + off
            return 0
        lax.fori_loop(0, 8, _al, 0)
        return 0
    lax.fori_loop(0, TPR, _addoff, 0)
    plsc.subcore_barrier()

    def _chunk(j, _):
        pltpu.async_copy(hs2.at[rowv.at[j]], dbuf, sem).wait()
        pltpu.sync_copy(dbuf, acc.at[colv.at[j]], add=True)
        return 0
    lax.fori_loop(0, TPR, _chunk, 0)
    plsc.subcore_barrier()

    pltpu.sync_copy(acc.at[pl.ds(s * 640, 640)],
                    g_o.at[pl.ds(c * NP + s * 640, 640)])


def _a0_body(x_ref, af_ref, wt_ref, dis_ref, o_ref):
    xb = x_ref[...]
    h0 = jnp.zeros((NBLK, EMB), F32)
    for f in range(9):
        oh = (xb[:, f][:, None]
              == lax.broadcasted_iota(I32, (1, 64), 1)).astype(F32)
        h0 = h0 + jnp.dot(oh, af_ref[pl.ds(f * 64, 64), :],
                          preferred_element_type=F32)
    hl = jnp.maximum(jnp.dot(h0, wt_ref[...], preferred_element_type=F32), 0.0)
    hs = hl * dis_ref[...]
    o_ref[0] = hs[:, :128]
    o_ref[1] = hs[:, 128:]


def _atom_layer0(x_p, atom_flat, w0t, dis):
    return pl.pallas_call(
        _a0_body,
        grid=(NGRID,),
        in_specs=[
            pl.BlockSpec((NBLK, 9), lambda i: (i, 0)),
            pl.BlockSpec((576, EMB), lambda i: (0, 0)),
            pl.BlockSpec((EMB, EMB), lambda i: (0, 0)),
            pl.BlockSpec((NBLK, 1), lambda i: (i, 0)),
        ],
        out_specs=pl.BlockSpec((2, NBLK, 128), lambda i: (0, i, 0)),
        out_shape=jax.ShapeDtypeStruct((2, NP, 128), F32),
    )(x_p, atom_flat, w0t, dis)


def _ep_body(g_ref, t_ref, bf_ref, hs_ref, dis_ref, sdeg_ref, dinv_ref,
             root_ref, pre_ref, stats_ref, sacc):
    i = pl.program_id(0)
    G = jnp.concatenate([g_ref[0], g_ref[1]], axis=1)
    HS = jnp.concatenate([hs_ref[0], hs_ref[1]], axis=1)
    Tb = t_ref[0] + t_ref[1]
    hl = HS * sdeg_ref[...]
    pre = (dis_ref[...]
           * (G + jnp.dot(Tb, bf_ref[...], preferred_element_type=F32))
           + (hl + root_ref[...]) * dinv_ref[...])
    pre_ref[...] = pre

    gid = i * NBLK + lax.broadcasted_iota(I32, (NBLK, 1), 0)
    pm = jnp.where(gid < N, pre, 0.0)

    @pl.when(i == 0)
    def _():
        sacc[...] = jnp.zeros((8, EMB), F32)

    sacc[0:1, :] = sacc[0:1, :] + jnp.sum(pm, axis=0, keepdims=True)
    sacc[1:2, :] = sacc[1:2, :] + jnp.sum(pm * pm, axis=0, keepdims=True)

    @pl.when(i == NGRID - 1)
    def _():
        stats_ref[...] = sacc[...]


def _epilogue(g3, t3, bondflat, hs3, dis, sdeg, dinv, root_l):
    return pl.pallas_call(
        _ep_body,
        grid=(NGRID,),
        in_specs=[
            pl.BlockSpec((2, NBLK, 128), lambda i: (0, i, 0)),
            pl.BlockSpec((2, NBLK, 48), lambda i: (0, i, 0)),
            pl.BlockSpec((48, EMB), lambda i: (0, 0)),
            pl.BlockSpec((2, NBLK, 128), lambda i: (0, i, 0)),
            pl.BlockSpec((NBLK, 1), lambda i: (i, 0)),
            pl.BlockSpec((NBLK, 1), lambda i: (i, 0)),
            pl.BlockSpec((NBLK, 1), lambda i: (i, 0)),
            pl.BlockSpec((1, EMB), lambda i: (0, 0)),
        ],
        out_specs=[
            pl.BlockSpec((NBLK, EMB), lambda i: (i, 0)),
            pl.BlockSpec((8, EMB), lambda i: (0, 0)),
        ],
        out_shape=[
            jax.ShapeDtypeStruct((NP, EMB), F32),
            jax.ShapeDtypeStruct((8, EMB), F32),
        ],
        scratch_shapes=[pltpu.VMEM((8, EMB), F32)],
    )(g3, t3, bondflat, hs3, dis, sdeg, dinv, root_l)


def _bn(stats_ref):
    mean = stats_ref[0:1, :] * (1.0 / N)
    ex2 = stats_ref[1:2, :] * (1.0 / N)
    var = ex2 - mean * mean
    return mean, lax.rsqrt(var + 1e-5)


def _mm_body(pre_ref, stats_ref, gm_ref, bt_ref, wt_ref, dis_ref, o_ref):
    mean, inv = _bn(stats_ref)
    h = (pre_ref[...] - mean) * inv * gm_ref[...] + bt_ref[...]
    h = jnp.maximum(h, 0.0)
    hl = jnp.maximum(jnp.dot(h, wt_ref[...], preferred_element_type=F32), 0.0)
    hs = hl * dis_ref[...]
    o_ref[0] = hs[:, :128]
    o_ref[1] = hs[:, 128:]


def _bn_layer(pre, stats, gamma_l, beta_l, wt, dis):
    return pl.pallas_call(
        _mm_body,
        grid=(NGRID,),
        in_specs=[
            pl.BlockSpec((NBLK, EMB), lambda i: (i, 0)),
            pl.BlockSpec((8, EMB), lambda i: (0, 0)),
            pl.BlockSpec((1, EMB), lambda i: (0, 0)),
            pl.BlockSpec((1, EMB), lambda i: (0, 0)),
            pl.BlockSpec((EMB, EMB), lambda i: (0, 0)),
            pl.BlockSpec((NBLK, 1), lambda i: (i, 0)),
        ],
        out_specs=pl.BlockSpec((2, NBLK, 128), lambda i: (0, i, 0)),
        out_shape=jax.ShapeDtypeStruct((2, NP, 128), F32),
    )(pre, stats, gamma_l, beta_l, wt, dis)


def _fin_body(pre_ref, stats_ref, gm_ref, bt_ref, o_ref):
    mean, inv = _bn(stats_ref)
    o_ref[...] = (pre_ref[...] - mean) * inv * gm_ref[...] + bt_ref[...]


def _bn_final(pre, stats, gamma_l, beta_l):
    return pl.pallas_call(
        _fin_body,
        grid=(NGRID,),
        in_specs=[
            pl.BlockSpec((NBLK, EMB), lambda i: (i, 0)),
            pl.BlockSpec((8, EMB), lambda i: (0, 0)),
            pl.BlockSpec((1, EMB), lambda i: (0, 0)),
            pl.BlockSpec((1, EMB), lambda i: (0, 0)),
        ],
        out_specs=pl.BlockSpec((NBLK, EMB), lambda i: (i, 0)),
        out_shape=jax.ShapeDtypeStruct((NP, EMB), F32),
    )(pre, stats, gamma_l, beta_l)


def kernel(x, edge_index, edge_attr, atom_tab, W, root, bond, gamma, beta):
    row = edge_index[0].astype(I32)
    col = edge_index[1].astype(I32)
    pad_e = EP - E
    pad_ids = (N + (jnp.arange(pad_e, dtype=I32) % (NP - N))).astype(I32)
    row2d = jnp.concatenate([row, pad_ids]).reshape(EROWS, ECH)
    col2d = jnp.concatenate([col, pad_ids]).reshape(EROWS, ECH)
    eap = jnp.concatenate(
        [edge_attr.astype(I32), jnp.zeros((pad_e, 3), I32)], axis=0)
    ea0 = eap[:, 0].reshape(EROWS, ECH)
    ea1 = eap[:, 1].reshape(EROWS, ECH)
    ea2 = eap[:, 2].reshape(EROWS, ECH)
    x_p = jnp.concatenate(
        [x.astype(I32), jnp.zeros((NP - N, x.shape[1]), I32)], axis=0)

    atom_flat = atom_tab.reshape(576, EMB)
    wts = [W[l].T for l in range(3)]
    bfs = [bond[l].reshape(48, EMB) for l in range(3)]

    dis, sdeg, dinv, t_flat = _sc_prep(row2d, col2d, ea0, ea1, ea2)
    dis = dis.reshape(NP, 1)
    sdeg = sdeg.reshape(NP, 1)
    dinv = dinv.reshape(NP, 1)
    t3 = t_flat.reshape(2, NP, 48)

    hs3 = _atom_layer0(x_p, atom_flat, wts[0], dis)
    out = None
    for l in range(3):
        g2 = _sc_spmm(hs3.reshape(2 * NP, 128), row2d, col2d)
        g3 = g2.reshape(2, NP, 128)
        pre, stats = _epilogue(g3, t3, bfs[l], hs3, dis, sdeg, dinv,
                               root[l][None, :])
        if l < 2:
            hs3 = _bn_layer(pre, stats, gamma[l][None, :], beta[l][None, :],
                            wts[l + 1], dis)
        else:
            out = _bn_final(pre, stats, gamma[l][None, :], beta[l][None, :])
    return out[:N]


# speedup vs baseline: 11.5049x; 11.5049x over previous
import functools

import jax
import jax.numpy as jnp
from jax import lax
from jax.experimental import pallas as pl
from jax.experimental.pallas import tpu as pltpu
from jax.experimental.pallas import tpu_sc as plsc

F32 = jnp.float32
I32 = jnp.int32

N = 10000
NP = 10240
E = 160000
EP = 163840
EMB = 256
NBLK = 1024
NGRID = NP // NBLK
ECH = 128
EROWS = EP // ECH
TPR = EROWS // 16
TPR2 = EROWS // 32

_MESH = plsc.VectorSubcoreMesh(core_axis_name="c", subcore_axis_name="s")


def _zero16():
    return jnp.zeros((16,), F32)


@functools.partial(
    pl.kernel,
    mesh=_MESH,
    compiler_params=pltpu.CompilerParams(needs_layout_passes=False),
    out_type=jax.ShapeDtypeStruct((2 * NP,), F32),
    scratch_types=[
        pltpu.VMEM_SHARED((NP,), F32),
        pltpu.VMEM((640,), F32),
        pltpu.VMEM((128,), F32),
        pltpu.VMEM((TPR2, ECH), I32),
    ],
)
def _sc_deg(row2d, deg_o, deg_sp, zb, ones, rowv):
    c = lax.axis_index("c")
    s = lax.axis_index("s")

    def _fill(i, _):
        zb[pl.ds(i * 16, 16)] = _zero16()
        return 0
    lax.fori_loop(0, 40, _fill, 0)

    def _fill1(i, _):
        ones[pl.ds(i * 16, 16)] = jnp.ones((16,), F32)
        return 0
    lax.fori_loop(0, 8, _fill1, 0)

    pltpu.sync_copy(zb, deg_sp.at[pl.ds(s * 640, 640)])
    plsc.subcore_barrier()

    base = c * (EROWS // 2) + s * TPR2
    pltpu.sync_copy(row2d.at[pl.ds(base, TPR2)], rowv)

    def _deg(j, _):
        pltpu.sync_copy(ones, deg_sp.at[rowv.at[j]], add=True)
        return 0
    lax.fori_loop(0, TPR2, _deg, 0)
    plsc.subcore_barrier()

    pltpu.sync_copy(deg_sp.at[pl.ds(s * 640, 640)],
                    deg_o.at[pl.ds(c * NP + s * 640, 640)])


def _dn_body(deg_ref, dis_ref, sdeg_ref, dinv_ref):
    degp = deg_ref[0] + deg_ref[1] + 1.0
    y = lax.rsqrt(degp)
    dis_ref[...] = y
    sdeg_ref[...] = degp * y
    dinv_ref[...] = y * y


def _deg_norm(deg2):
    return pl.pallas_call(
        _dn_body,
        grid=(NGRID,),
        in_specs=[pl.BlockSpec((2, NBLK, 1), lambda i: (0, i, 0))],
        out_specs=[
            pl.BlockSpec((NBLK, 1), lambda i: (i, 0)),
            pl.BlockSpec((NBLK, 1), lambda i: (i, 0)),
            pl.BlockSpec((NBLK, 1), lambda i: (i, 0)),
        ],
        out_shape=[
            jax.ShapeDtypeStruct((NP, 1), F32),
            jax.ShapeDtypeStruct((NP, 1), F32),
            jax.ShapeDtypeStruct((NP, 1), F32),
        ],
    )(deg2)


@functools.partial(
    pl.kernel,
    mesh=_MESH,
    compiler_params=pltpu.CompilerParams(needs_layout_passes=False),
    out_type=jax.ShapeDtypeStruct((2 * NP * 48,), F32),
    scratch_types=[
        pltpu.VMEM_SHARED((NP * 48,), F32),
        pltpu.VMEM((3840,), F32),
        pltpu.VMEM((NP,), F32),
        pltpu.VMEM((TPR2, ECH), I32),
        pltpu.VMEM((TPR2, ECH), I32),
        pltpu.VMEM((TPR2, ECH), I32),
        pltpu.VMEM((TPR2, ECH), I32),
        pltpu.VMEM((TPR2, ECH), I32),
        pltpu.VMEM((128,), F32),
        pltpu.VMEM((8, ECH), I32),
    ],
)
def _sc_tbins(row2d, col2d, ea0, ea1, ea2, dis_h,
              t_o, t_sp, zb, disv, rowt, colt, e0v, e1v, e2v, valb, idxb):
    c = lax.axis_index("c")
    s = lax.axis_index("s")

    def _fill(i, _):
        zb[pl.ds(i * 16, 16)] = _zero16()
        return 0
    lax.fori_loop(0, 240, _fill, 0)

    def _zt(i, _):
        pltpu.sync_copy(zb, t_sp.at[pl.ds(s * 30720 + i * 3840, 3840)])
        return 0
    lax.fori_loop(0, 8, _zt, 0)
    plsc.subcore_barrier()

    pltpu.sync_copy(dis_h, disv)

    base = c * (EROWS // 2) + s * TPR2
    pltpu.sync_copy(row2d.at[pl.ds(base, TPR2)], rowt)
    pltpu.sync_copy(col2d.at[pl.ds(base, TPR2)], colt)
    pltpu.sync_copy(ea0.at[pl.ds(base, TPR2)], e0v)
    pltpu.sync_copy(ea1.at[pl.ds(base, TPR2)], e1v)
    pltpu.sync_copy(ea2.at[pl.ds(base, TPR2)], e2v)

    def _tchunk(j, _):
        def _lane(k, _):
            sl = pl.ds(k * 16, 16)
            ridx = rowt.at[j][sl]
            valb[sl] = plsc.load_gather(disv, [ridx])
            cv = colt.at[j][sl] * 48
            idxb[0, sl] = cv + e0v.at[j][sl]
            idxb[1, sl] = cv + 16 + e1v.at[j][sl]
            idxb[2, sl] = cv + 32 + e2v.at[j][sl]
            return 0
        lax.fori_loop(0, 8, _lane, 0)
        pltpu.sync_copy(valb, t_sp.at[idxb.at[0]], add=True)
        pltpu.sync_copy(valb, t_sp.at[idxb.at[1]], add=True)
        pltpu.sync_copy(valb, t_sp.at[idxb.at[2]], add=True)
        return 0
    lax.fori_loop(0, TPR2, _tchunk, 0)
    plsc.subcore_barrier()

    pltpu.sync_copy(t_sp.at[pl.ds(s * 30720, 30720)],
                    t_o.at[pl.ds(c * (NP * 48) + s * 30720, 30720)])


@functools.partial(
    pl.kernel,
    mesh=_MESH,
    compiler_params=pltpu.CompilerParams(needs_layout_passes=False),
    out_type=jax.ShapeDtypeStruct((2 * NP, 128), F32),
    scratch_types=[
        pltpu.VMEM_SHARED((NP, 128), F32),
        pltpu.VMEM((TPR, ECH), I32),
        pltpu.VMEM((TPR, ECH), I32),
        pltpu.VMEM((ECH, 128), F32),
        pltpu.VMEM((64, 128), F32),
        pltpu.SemaphoreType.DMA,
    ],
)
def _sc_spmm(hs2, row2d, col2d, g_o, acc, rowv, colv, dbuf, zbuf, sem):
    c = lax.axis_index("c")
    s = lax.axis_index("s")

    def _zrow(r, _):
        rr = zbuf.at[r]

        def _zl(k, _):
            rr[pl.ds(k * 16, 16)] = _zero16()
            return 0
        lax.fori_loop(0, 8, _zl, 0)
        return 0
    lax.fori_loop(0, 64, _zrow, 0)

    def _zacc(i, _):
        pltpu.sync_copy(zbuf, acc.at[pl.ds(s * 640 + i * 64, 64)])
        return 0
    lax.fori_loop(0, 10, _zacc, 0)

    pltpu.sync_copy(row2d.at[pl.ds(s * TPR, TPR)], rowv)
    pltpu.sync_copy(col2d.at[pl.ds(s * TPR, TPR)], colv)

    off = c * NP

    def _addoff(j, _):
        rr = rowv.at[j]

        def _al(k, _):
            sl = pl.ds(k * 16, 16)
            rr[sl] = rr[sl] + off
            return 0
        lax.fori_loop(0, 8, _al, 0)
        return 0
    lax.fori_loop(0, TPR, _addoff, 0)
    plsc.subcore_barrier()

    def _chunk(j, _):
        pltpu.async_copy(hs2.at[rowv.at[j]], dbuf, sem).wait()
        pltpu.sync_copy(dbuf, acc.at[colv.at[j]], add=True)
        return 0
    lax.fori_loop(0, TPR, _chunk, 0)
    plsc.subcore_barrier()

    pltpu.sync_copy(acc.at[pl.ds(s * 640, 640)],
                    g_o.at[pl.ds(c * NP + s * 640, 640)])


def _a0_body(x_ref, af_ref, wt_ref, dis_ref, o_ref):
    xb = x_ref[...]
    h0 = jnp.zeros((NBLK, EMB), F32)
    for f in range(9):
        oh = (xb[:, f][:, None]
              == lax.broadcasted_iota(I32, (1, 64), 1)).astype(F32)
        h0 = h0 + jnp.dot(oh, af_ref[pl.ds(f * 64, 64), :],
                          preferred_element_type=F32)
    hl = jnp.maximum(jnp.dot(h0, wt_ref[...], preferred_element_type=F32), 0.0)
    hs = hl * dis_ref[...]
    o_ref[0] = hs[:, :128]
    o_ref[1] = hs[:, 128:]


def _atom_layer0(x_p, atom_flat, w0t, dis):
    return pl.pallas_call(
        _a0_body,
        grid=(NGRID,),
        in_specs=[
            pl.BlockSpec((NBLK, 9), lambda i: (i, 0)),
            pl.BlockSpec((576, EMB), lambda i: (0, 0)),
            pl.BlockSpec((EMB, EMB), lambda i: (0, 0)),
            pl.BlockSpec((NBLK, 1), lambda i: (i, 0)),
        ],
        out_specs=pl.BlockSpec((2, NBLK, 128), lambda i: (0, i, 0)),
        out_shape=jax.ShapeDtypeStruct((2, NP, 128), F32),
    )(x_p, atom_flat, w0t, dis)


def _ep_body(g_ref, t_ref, bf_ref, hs_ref, dis_ref, sdeg_ref, dinv_ref,
             root_ref, pre_ref, stats_ref, sacc):
    i = pl.program_id(0)
    G = jnp.concatenate([g_ref[0], g_ref[1]], axis=1)
    HS = jnp.concatenate([hs_ref[0], hs_ref[1]], axis=1)
    Tb = t_ref[0] + t_ref[1]
    hl = HS * sdeg_ref[...]
    pre = (dis_ref[...]
           * (G + jnp.dot(Tb, bf_ref[...], preferred_element_type=F32))
           + (hl + root_ref[...]) * dinv_ref[...])
    pre_ref[...] = pre

    gid = i * NBLK + lax.broadcasted_iota(I32, (NBLK, 1), 0)
    pm = jnp.where(gid < N, pre, 0.0)

    @pl.when(i == 0)
    def _():
        sacc[...] = jnp.zeros((8, EMB), F32)

    sacc[0:1, :] = sacc[0:1, :] + jnp.sum(pm, axis=0, keepdims=True)
    sacc[1:2, :] = sacc[1:2, :] + jnp.sum(pm * pm, axis=0, keepdims=True)

    @pl.when(i == NGRID - 1)
    def _():
        stats_ref[...] = sacc[...]


def _epilogue(g3, t3, bondflat, hs3, dis, sdeg, dinv, root_l):
    return pl.pallas_call(
        _ep_body,
        grid=(NGRID,),
        in_specs=[
            pl.BlockSpec((2, NBLK, 128), lambda i: (0, i, 0)),
            pl.BlockSpec((2, NBLK, 48), lambda i: (0, i, 0)),
            pl.BlockSpec((48, EMB), lambda i: (0, 0)),
            pl.BlockSpec((2, NBLK, 128), lambda i: (0, i, 0)),
            pl.BlockSpec((NBLK, 1), lambda i: (i, 0)),
            pl.BlockSpec((NBLK, 1), lambda i: (i, 0)),
            pl.BlockSpec((NBLK, 1), lambda i: (i, 0)),
            pl.BlockSpec((1, EMB), lambda i: (0, 0)),
        ],
        out_specs=[
            pl.BlockSpec((NBLK, EMB), lambda i: (i, 0)),
            pl.BlockSpec((8, EMB), lambda i: (0, 0)),
        ],
        out_shape=[
            jax.ShapeDtypeStruct((NP, EMB), F32),
            jax.ShapeDtypeStruct((8, EMB), F32),
        ],
        scratch_shapes=[pltpu.VMEM((8, EMB), F32)],
    )(g3, t3, bondflat, hs3, dis, sdeg, dinv, root_l)


def _bn(stats_ref):
    mean = stats_ref[0:1, :] * (1.0 / N)
    ex2 = stats_ref[1:2, :] * (1.0 / N)
    var = ex2 - mean * mean
    return mean, lax.rsqrt(var + 1e-5)


def _mm_body(pre_ref, stats_ref, gm_ref, bt_ref, wt_ref, dis_ref, o_ref):
    mean, inv = _bn(stats_ref)
    h = (pre_ref[...] - mean) * inv * gm_ref[...] + bt_ref[...]
    h = jnp.maximum(h, 0.0)
    hl = jnp.maximum(jnp.dot(h, wt_ref[...], preferred_element_type=F32), 0.0)
    hs = hl * dis_ref[...]
    o_ref[0] = hs[:, :128]
    o_ref[1] = hs[:, 128:]


def _bn_layer(pre, stats, gamma_l, beta_l, wt, dis):
    return pl.pallas_call(
        _mm_body,
        grid=(NGRID,),
        in_specs=[
            pl.BlockSpec((NBLK, EMB), lambda i: (i, 0)),
            pl.BlockSpec((8, EMB), lambda i: (0, 0)),
            pl.BlockSpec((1, EMB), lambda i: (0, 0)),
            pl.BlockSpec((1, EMB), lambda i: (0, 0)),
            pl.BlockSpec((EMB, EMB), lambda i: (0, 0)),
            pl.BlockSpec((NBLK, 1), lambda i: (i, 0)),
        ],
        out_specs=pl.BlockSpec((2, NBLK, 128), lambda i: (0, i, 0)),
        out_shape=jax.ShapeDtypeStruct((2, NP, 128), F32),
    )(pre, stats, gamma_l, beta_l, wt, dis)


def _fin_body(pre_ref, stats_ref, gm_ref, bt_ref, o_ref):
    mean, inv = _bn(stats_ref)
    o_ref[...] = (pre_ref[...] - mean) * inv * gm_ref[...] + bt_ref[...]


def _bn_final(pre, stats, gamma_l, beta_l):
    return pl.pallas_call(
        _fin_body,
        grid=(NGRID,),
        in_specs=[
            pl.BlockSpec((NBLK, EMB), lambda i: (i, 0)),
            pl.BlockSpec((8, EMB), lambda i: (0, 0)),
            pl.BlockSpec((1, EMB), lambda i: (0, 0)),
            pl.BlockSpec((1, EMB), lambda i: (0, 0)),
        ],
        out_specs=pl.BlockSpec((NBLK, EMB), lambda i: (i, 0)),
        out_shape=jax.ShapeDtypeStruct((NP, EMB), F32),
    )(pre, stats, gamma_l, beta_l)


def kernel(x, edge_index, edge_attr, atom_tab, W, root, bond, gamma, beta):
    row = edge_index[0].astype(I32)
    col = edge_index[1].astype(I32)
    pad_e = EP - E
    pad_ids = (N + (jnp.arange(pad_e, dtype=I32) % (NP - N))).astype(I32)
    row2d = jnp.concatenate([row, pad_ids]).reshape(EROWS, ECH)
    col2d = jnp.concatenate([col, pad_ids]).reshape(EROWS, ECH)
    eap = jnp.concatenate(
        [edge_attr.astype(I32), jnp.zeros((pad_e, 3), I32)], axis=0)
    ea0 = eap[:, 0].reshape(EROWS, ECH)
    ea1 = eap[:, 1].reshape(EROWS, ECH)
    ea2 = eap[:, 2].reshape(EROWS, ECH)
    x_p = jnp.concatenate(
        [x.astype(I32), jnp.zeros((NP - N, x.shape[1]), I32)], axis=0)

    atom_flat = atom_tab.reshape(576, EMB)
    wts = [W[l].T for l in range(3)]
    bfs = [bond[l].reshape(48, EMB) for l in range(3)]

    deg2 = _sc_deg(row2d).reshape(2, NP, 1)
    dis, sdeg, dinv = _deg_norm(deg2)
    t_flat = _sc_tbins(row2d, col2d, ea0, ea1, ea2, dis.reshape(NP))
    t3 = t_flat.reshape(2, NP, 48)

    hs3 = _atom_layer0(x_p, atom_flat, wts[0], dis)
    out = None
    for l in range(3):
        g2 = _sc_spmm(hs3.reshape(2 * NP, 128), row2d, col2d)
        g3 = g2.reshape(2, NP, 128)
        pre, stats = _epilogue(g3, t3, bfs[l], hs3, dis, sdeg, dinv,
                               root[l][None, :])
        if l < 2:
            hs3 = _bn_layer(pre, stats, gamma[l][None, :], beta[l][None, :],
                            wts[l + 1], dis)
        else:
            out = _bn_final(pre, stats, gamma[l][None, :], beta[l][None, :])
    return out[:N]


# speedup vs baseline: 15.8185x; 1.3749x over previous
import functools

import jax
import jax.numpy as jnp
from jax import lax
from jax.experimental import pallas as pl
from jax.experimental.pallas import tpu as pltpu
from jax.experimental.pallas import tpu_sc as plsc

F32 = jnp.float32
I32 = jnp.int32

N = 10000
NP = 10240
E = 160000
EP = 163840
EMB = 256
NBLK = 1024
NGRID = NP // NBLK
ECH = 128
EROWS = EP // ECH
TPR = EROWS // 16
TPR2 = EROWS // 32

_MESH = plsc.VectorSubcoreMesh(core_axis_name="c", subcore_axis_name="s")


def _zero16():
    return jnp.zeros((16,), F32)


@functools.partial(
    pl.kernel,
    mesh=_MESH,
    compiler_params=pltpu.CompilerParams(needs_layout_passes=False),
    out_type=jax.ShapeDtypeStruct((2 * NP,), F32),
    scratch_types=[
        pltpu.VMEM_SHARED((NP,), F32),
        pltpu.VMEM((640,), F32),
        pltpu.VMEM((128,), F32),
        pltpu.VMEM((TPR2, ECH), I32),
    ],
)
def _sc_deg(row2d, deg_o, deg_sp, zb, ones, rowv):
    c = lax.axis_index("c")
    s = lax.axis_index("s")

    def _fill(i, _):
        zb[pl.ds(i * 16, 16)] = _zero16()
        return 0
    lax.fori_loop(0, 40, _fill, 0)

    def _fill1(i, _):
        ones[pl.ds(i * 16, 16)] = jnp.ones((16,), F32)
        return 0
    lax.fori_loop(0, 8, _fill1, 0)

    pltpu.sync_copy(zb, deg_sp.at[pl.ds(s * 640, 640)])
    plsc.subcore_barrier()

    base = c * (EROWS // 2) + s * TPR2
    pltpu.sync_copy(row2d.at[pl.ds(base, TPR2)], rowv)

    def _deg(j, _):
        pltpu.sync_copy(ones, deg_sp.at[rowv.at[j]], add=True)
        return 0
    lax.fori_loop(0, TPR2, _deg, 0)
    plsc.subcore_barrier()

    pltpu.sync_copy(deg_sp.at[pl.ds(s * 640, 640)],
                    deg_o.at[pl.ds(c * NP + s * 640, 640)])


def _dn_body(deg_ref, dis_ref, sdeg_ref, dinv_ref):
    degp = deg_ref[0] + deg_ref[1] + 1.0
    y = lax.rsqrt(degp)
    dis_ref[...] = y
    sdeg_ref[...] = degp * y
    dinv_ref[...] = y * y


def _deg_norm(deg2):
    return pl.pallas_call(
        _dn_body,
        grid=(NGRID,),
        in_specs=[pl.BlockSpec((2, NBLK, 1), lambda i: (0, i, 0))],
        out_specs=[
            pl.BlockSpec((NBLK, 1), lambda i: (i, 0)),
            pl.BlockSpec((NBLK, 1), lambda i: (i, 0)),
            pl.BlockSpec((NBLK, 1), lambda i: (i, 0)),
        ],
        out_shape=[
            jax.ShapeDtypeStruct((NP, 1), F32),
            jax.ShapeDtypeStruct((NP, 1), F32),
            jax.ShapeDtypeStruct((NP, 1), F32),
        ],
    )(deg2)


@functools.partial(
    pl.kernel,
    mesh=_MESH,
    compiler_params=pltpu.CompilerParams(needs_layout_passes=False),
    out_type=jax.ShapeDtypeStruct((2 * NP * 48,), F32),
    scratch_types=[
        pltpu.VMEM_SHARED((NP * 48,), F32),
        pltpu.VMEM((3840,), F32),
        pltpu.VMEM((NP,), F32),
        pltpu.VMEM((TPR2, ECH), I32),
        pltpu.VMEM((TPR2, ECH), I32),
        pltpu.VMEM((TPR2, ECH), I32),
        pltpu.VMEM((TPR2, ECH), I32),
        pltpu.VMEM((TPR2, ECH), I32),
        pltpu.VMEM((128,), F32),
        pltpu.VMEM((8, ECH), I32),
    ],
)
def _sc_tbins(row2d, col2d, ea0, ea1, ea2, dis_h,
              t_o, t_sp, zb, disv, rowt, colt, e0v, e1v, e2v, valb, idxb):
    c = lax.axis_index("c")
    s = lax.axis_index("s")

    def _fill(i, _):
        zb[pl.ds(i * 16, 16)] = _zero16()
        return 0
    lax.fori_loop(0, 240, _fill, 0)

    def _zt(i, _):
        pltpu.sync_copy(zb, t_sp.at[pl.ds(s * 30720 + i * 3840, 3840)])
        return 0
    lax.fori_loop(0, 8, _zt, 0)
    plsc.subcore_barrier()

    pltpu.sync_copy(dis_h, disv)

    base = c * (EROWS // 2) + s * TPR2
    pltpu.sync_copy(row2d.at[pl.ds(base, TPR2)], rowt)
    pltpu.sync_copy(col2d.at[pl.ds(base, TPR2)], colt)
    pltpu.sync_copy(ea0.at[pl.ds(base, TPR2)], e0v)
    pltpu.sync_copy(ea1.at[pl.ds(base, TPR2)], e1v)
    pltpu.sync_copy(ea2.at[pl.ds(base, TPR2)], e2v)

    def _tchunk(j, _):
        def _lane(k, _):
            sl = pl.ds(k * 16, 16)
            ridx = rowt.at[j][sl]
            valb[sl] = plsc.load_gather(disv, [ridx])
            cv = colt.at[j][sl] * 48
            idxb[0, sl] = cv + e0v.at[j][sl]
            idxb[1, sl] = cv + 16 + e1v.at[j][sl]
            idxb[2, sl] = cv + 32 + e2v.at[j][sl]
            return 0
        lax.fori_loop(0, 8, _lane, 0)
        pltpu.sync_copy(valb, t_sp.at[idxb.at[0]], add=True)
        pltpu.sync_copy(valb, t_sp.at[idxb.at[1]], add=True)
        pltpu.sync_copy(valb, t_sp.at[idxb.at[2]], add=True)
        return 0
    lax.fori_loop(0, TPR2, _tchunk, 0)
    plsc.subcore_barrier()

    pltpu.sync_copy(t_sp.at[pl.ds(s * 30720, 30720)],
                    t_o.at[pl.ds(c * (NP * 48) + s * 30720, 30720)])


SCH = 64
SROWS = EP // SCH
SPR = SROWS // 16


@functools.partial(
    pl.kernel,
    mesh=_MESH,
    compiler_params=pltpu.CompilerParams(needs_layout_passes=False),
    out_type=jax.ShapeDtypeStruct((2 * NP, 128), F32),
    scratch_types=[
        pltpu.VMEM_SHARED((NP, 128), F32),
        pltpu.VMEM((SPR // 2, SCH), I32),
        pltpu.VMEM((SPR // 2, SCH), I32),
        pltpu.VMEM((SCH, 128), F32),
        pltpu.VMEM((SCH, 128), F32),
        pltpu.VMEM((SCH, 128), F32),
        pltpu.SemaphoreType.DMA,
        pltpu.SemaphoreType.DMA,
        pltpu.SemaphoreType.DMA,
        pltpu.SemaphoreType.DMA,
        pltpu.SemaphoreType.DMA,
        pltpu.SemaphoreType.DMA,
    ],
)
def _sc_spmm(hs2, row2d, col2d, g_o, acc, rowv, colv,
             db0, db1, db2, gs0, gs1, gs2, ss0, ss1, ss2):
    dbufs = (db0, db1, db2)
    gsems = (gs0, gs1, gs2)
    ssems = (ss0, ss1, ss2)
    c = lax.axis_index("c")
    s = lax.axis_index("s")
    hpr = SPR // 2

    def _zl(k, _):
        rr = db0.at[k]

        def _zi(i, _):
            rr[pl.ds(i * 16, 16)] = _zero16()
            return 0
        lax.fori_loop(0, 8, _zi, 0)
        return 0
    lax.fori_loop(0, SCH, _zl, 0)

    def _zacc(i, _):
        pltpu.sync_copy(db0, acc.at[pl.ds(s * 640 + i * SCH, SCH)])
        return 0
    lax.fori_loop(0, 640 // SCH, _zacc, 0)
    plsc.subcore_barrier()

    off = c * NP
    for h in range(2):
        base = s * SPR + h * hpr
        pltpu.sync_copy(row2d.at[pl.ds(base, hpr)], rowv)
        pltpu.sync_copy(col2d.at[pl.ds(base, hpr)], colv)

        def _addoff(j, _):
            rr = rowv.at[j]

            def _al(k, _):
                sl = pl.ds(k * 16, 16)
                rr[sl] = rr[sl] + off
                return 0
            lax.fori_loop(0, SCH // 16, _al, 0)
            return 0
        lax.fori_loop(0, hpr, _addoff, 0)

        def _steps(jj, _):
            for b in range(3):
                t = 3 * jj + b
                b2 = (b + 2) % 3

                @pl.when(t < hpr)
                def _():
                    @pl.when(t >= 3)
                    def _():
                        pltpu.make_async_copy(
                            dbufs[b], acc.at[colv.at[t]], ssems[b]).wait()
                    pltpu.async_copy(hs2.at[rowv.at[t]], dbufs[b], gsems[b])

                @pl.when((t >= 1) & (t < hpr + 1))
                def _():
                    pltpu.make_async_copy(
                        hs2.at[rowv.at[jnp.maximum(t - 1, 0)]],
                        dbufs[b2], gsems[b2]).wait()
                    pltpu.async_copy(dbufs[b2],
                                     acc.at[colv.at[jnp.maximum(t - 1, 0)]],
                                     ssems[b2], add=True)
            return 0
        lax.fori_loop(0, (hpr + 1 + 2) // 3, _steps, 0)

        for b in range(3):
            pltpu.make_async_copy(dbufs[b], acc.at[colv.at[hpr - 3 + b]],
                                  ssems[b]).wait()
    plsc.subcore_barrier()

    pltpu.sync_copy(acc.at[pl.ds(s * 640, 640)],
                    g_o.at[pl.ds(c * NP + s * 640, 640)])


def _a0_body(x_ref, af_ref, wt_ref, dis_ref, o_ref):
    xb = x_ref[...]
    h0 = jnp.zeros((NBLK, EMB), F32)
    for f in range(9):
        oh = (xb[:, f][:, None]
              == lax.broadcasted_iota(I32, (1, 64), 1)).astype(F32)
        h0 = h0 + jnp.dot(oh, af_ref[pl.ds(f * 64, 64), :],
                          preferred_element_type=F32)
    hl = jnp.maximum(jnp.dot(h0, wt_ref[...], preferred_element_type=F32), 0.0)
    hs = hl * dis_ref[...]
    o_ref[0] = hs[:, :128]
    o_ref[1] = hs[:, 128:]


def _atom_layer0(x_p, atom_flat, w0t, dis):
    return pl.pallas_call(
        _a0_body,
        grid=(NGRID,),
        in_specs=[
            pl.BlockSpec((NBLK, 9), lambda i: (i, 0)),
            pl.BlockSpec((576, EMB), lambda i: (0, 0)),
            pl.BlockSpec((EMB, EMB), lambda i: (0, 0)),
            pl.BlockSpec((NBLK, 1), lambda i: (i, 0)),
        ],
        out_specs=pl.BlockSpec((2, NBLK, 128), lambda i: (0, i, 0)),
        out_shape=jax.ShapeDtypeStruct((2, NP, 128), F32),
    )(x_p, atom_flat, w0t, dis)


def _ep_body(g_ref, t_ref, bf_ref, hs_ref, dis_ref, sdeg_ref, dinv_ref,
             root_ref, pre_ref, stats_ref, sacc):
    i = pl.program_id(0)
    G = jnp.concatenate([g_ref[0], g_ref[1]], axis=1)
    HS = jnp.concatenate([hs_ref[0], hs_ref[1]], axis=1)
    Tb = t_ref[0] + t_ref[1]
    hl = HS * sdeg_ref[...]
    pre = (dis_ref[...]
           * (G + jnp.dot(Tb, bf_ref[...], preferred_element_type=F32))
           + (hl + root_ref[...]) * dinv_ref[...])
    pre_ref[...] = pre

    gid = i * NBLK + lax.broadcasted_iota(I32, (NBLK, 1), 0)
    pm = jnp.where(gid < N, pre, 0.0)

    @pl.when(i == 0)
    def _():
        sacc[...] = jnp.zeros((8, EMB), F32)

    sacc[0:1, :] = sacc[0:1, :] + jnp.sum(pm, axis=0, keepdims=True)
    sacc[1:2, :] = sacc[1:2, :] + jnp.sum(pm * pm, axis=0, keepdims=True)

    @pl.when(i == NGRID - 1)
    def _():
        stats_ref[...] = sacc[...]


def _epilogue(g3, t3, bondflat, hs3, dis, sdeg, dinv, root_l):
    return pl.pallas_call(
        _ep_body,
        grid=(NGRID,),
        in_specs=[
            pl.BlockSpec((2, NBLK, 128), lambda i: (0, i, 0)),
            pl.BlockSpec((2, NBLK, 48), lambda i: (0, i, 0)),
            pl.BlockSpec((48, EMB), lambda i: (0, 0)),
            pl.BlockSpec((2, NBLK, 128), lambda i: (0, i, 0)),
            pl.BlockSpec((NBLK, 1), lambda i: (i, 0)),
            pl.BlockSpec((NBLK, 1), lambda i: (i, 0)),
            pl.BlockSpec((NBLK, 1), lambda i: (i, 0)),
            pl.BlockSpec((1, EMB), lambda i: (0, 0)),
        ],
        out_specs=[
            pl.BlockSpec((NBLK, EMB), lambda i: (i, 0)),
            pl.BlockSpec((8, EMB), lambda i: (0, 0)),
        ],
        out_shape=[
            jax.ShapeDtypeStruct((NP, EMB), F32),
            jax.ShapeDtypeStruct((8, EMB), F32),
        ],
        scratch_shapes=[pltpu.VMEM((8, EMB), F32)],
    )(g3, t3, bondflat, hs3, dis, sdeg, dinv, root_l)


def _bn(stats_ref):
    mean = stats_ref[0:1, :] * (1.0 / N)
    ex2 = stats_ref[1:2, :] * (1.0 / N)
    var = ex2 - mean * mean
    return mean, lax.rsqrt(var + 1e-5)


def _mm_body(pre_ref, stats_ref, gm_ref, bt_ref, wt_ref, dis_ref, o_ref):
    mean, inv = _bn(stats_ref)
    h = (pre_ref[...] - mean) * inv * gm_ref[...] + bt_ref[...]
    h = jnp.maximum(h, 0.0)
    hl = jnp.maximum(jnp.dot(h, wt_ref[...], preferred_element_type=F32), 0.0)
    hs = hl * dis_ref[...]
    o_ref[0] = hs[:, :128]
    o_ref[1] = hs[:, 128:]


def _bn_layer(pre, stats, gamma_l, beta_l, wt, dis):
    return pl.pallas_call(
        _mm_body,
        grid=(NGRID,),
        in_specs=[
            pl.BlockSpec((NBLK, EMB), lambda i: (i, 0)),
            pl.BlockSpec((8, EMB), lambda i: (0, 0)),
            pl.BlockSpec((1, EMB), lambda i: (0, 0)),
            pl.BlockSpec((1, EMB), lambda i: (0, 0)),
            pl.BlockSpec((EMB, EMB), lambda i: (0, 0)),
            pl.BlockSpec((NBLK, 1), lambda i: (i, 0)),
        ],
        out_specs=pl.BlockSpec((2, NBLK, 128), lambda i: (0, i, 0)),
        out_shape=jax.ShapeDtypeStruct((2, NP, 128), F32),
    )(pre, stats, gamma_l, beta_l, wt, dis)


def _fin_body(pre_ref, stats_ref, gm_ref, bt_ref, o_ref):
    mean, inv = _bn(stats_ref)
    o_ref[...] = (pre_ref[...] - mean) * inv * gm_ref[...] + bt_ref[...]


def _bn_final(pre, stats, gamma_l, beta_l):
    return pl.pallas_call(
        _fin_body,
        grid=(NGRID,),
        in_specs=[
            pl.BlockSpec((NBLK, EMB), lambda i: (i, 0)),
            pl.BlockSpec((8, EMB), lambda i: (0, 0)),
            pl.BlockSpec((1, EMB), lambda i: (0, 0)),
            pl.BlockSpec((1, EMB), lambda i: (0, 0)),
        ],
        out_specs=pl.BlockSpec((NBLK, EMB), lambda i: (i, 0)),
        out_shape=jax.ShapeDtypeStruct((NP, EMB), F32),
    )(pre, stats, gamma_l, beta_l)


def kernel(x, edge_index, edge_attr, atom_tab, W, root, bond, gamma, beta):
    row = edge_index[0].astype(I32)
    col = edge_index[1].astype(I32)
    pad_e = EP - E
    pad_ids = (N + (jnp.arange(pad_e, dtype=I32) % (NP - N))).astype(I32)
    row2d = jnp.concatenate([row, pad_ids]).reshape(EROWS, ECH)
    col2d = jnp.concatenate([col, pad_ids]).reshape(EROWS, ECH)
    eap = jnp.concatenate(
        [edge_attr.astype(I32), jnp.zeros((pad_e, 3), I32)], axis=0)
    ea0 = eap[:, 0].reshape(EROWS, ECH)
    ea1 = eap[:, 1].reshape(EROWS, ECH)
    ea2 = eap[:, 2].reshape(EROWS, ECH)
    x_p = jnp.concatenate(
        [x.astype(I32), jnp.zeros((NP - N, x.shape[1]), I32)], axis=0)

    atom_flat = atom_tab.reshape(576, EMB)
    wts = [W[l].T for l in range(3)]
    bfs = [bond[l].reshape(48, EMB) for l in range(3)]

    deg2 = _sc_deg(row2d).reshape(2, NP, 1)
    dis, sdeg, dinv = _deg_norm(deg2)
    t_flat = _sc_tbins(row2d, col2d, ea0, ea1, ea2, dis.reshape(NP))
    t3 = t_flat.reshape(2, NP, 48)

    hs3 = _atom_layer0(x_p, atom_flat, wts[0], dis)
    out = None
    for l in range(3):
        g2 = _sc_spmm(hs3.reshape(2 * NP, 128),
                      row2d.reshape(SROWS, SCH), col2d.reshape(SROWS, SCH))
        g3 = g2.reshape(2, NP, 128)
        pre, stats = _epilogue(g3, t3, bfs[l], hs3, dis, sdeg, dinv,
                               root[l][None, :])
        if l < 2:
            hs3 = _bn_layer(pre, stats, gamma[l][None, :], beta[l][None, :],
                            wts[l + 1], dis)
        else:
            out = _bn_final(pre, stats, gamma[l][None, :], beta[l][None, :])
    return out[:N]


# speedup vs baseline: 16.6421x; 1.0521x over previous
import functools

import jax
import jax.numpy as jnp
from jax import lax
from jax.experimental import pallas as pl
from jax.experimental.pallas import tpu as pltpu
from jax.experimental.pallas import tpu_sc as plsc

F32 = jnp.float32
I32 = jnp.int32

N = 10000
NP = 10240
E = 160000
EP = 163840
EMB = 256
NBLK = 1024
NGRID = NP // NBLK
ECH = 128
EROWS = EP // ECH
TPR = EROWS // 16
TPR2 = EROWS // 32

_MESH = plsc.VectorSubcoreMesh(core_axis_name="c", subcore_axis_name="s")


def _zero16():
    return jnp.zeros((16,), F32)


@functools.partial(
    pl.kernel,
    mesh=_MESH,
    compiler_params=pltpu.CompilerParams(needs_layout_passes=False),
    out_type=jax.ShapeDtypeStruct((2 * NP,), F32),
    scratch_types=[
        pltpu.VMEM_SHARED((NP,), F32),
        pltpu.VMEM((640,), F32),
        pltpu.VMEM((128,), F32),
        pltpu.VMEM((TPR2, ECH), I32),
    ],
)
def _sc_deg(row2d, deg_o, deg_sp, zb, ones, rowv):
    c = lax.axis_index("c")
    s = lax.axis_index("s")

    def _fill(i, _):
        zb[pl.ds(i * 16, 16)] = _zero16()
        return 0
    lax.fori_loop(0, 40, _fill, 0)

    def _fill1(i, _):
        ones[pl.ds(i * 16, 16)] = jnp.ones((16,), F32)
        return 0
    lax.fori_loop(0, 8, _fill1, 0)

    pltpu.sync_copy(zb, deg_sp.at[pl.ds(s * 640, 640)])
    plsc.subcore_barrier()

    base = c * (EROWS // 2) + s * TPR2
    pltpu.sync_copy(row2d.at[pl.ds(base, TPR2)], rowv)

    def _deg(j, _):
        pltpu.sync_copy(ones, deg_sp.at[rowv.at[j]], add=True)
        return 0
    lax.fori_loop(0, TPR2, _deg, 0)
    plsc.subcore_barrier()

    pltpu.sync_copy(deg_sp.at[pl.ds(s * 640, 640)],
                    deg_o.at[pl.ds(c * NP + s * 640, 640)])


def _dn_body(deg_ref, dis_ref, sdeg_ref, dinv_ref):
    degp = deg_ref[0] + deg_ref[1] + 1.0
    y = lax.rsqrt(degp)
    dis_ref[...] = y
    sdeg_ref[...] = degp * y
    dinv_ref[...] = y * y


def _deg_norm(deg2):
    return pl.pallas_call(
        _dn_body,
        grid=(NGRID,),
        in_specs=[pl.BlockSpec((2, NBLK, 1), lambda i: (0, i, 0))],
        out_specs=[
            pl.BlockSpec((NBLK, 1), lambda i: (i, 0)),
            pl.BlockSpec((NBLK, 1), lambda i: (i, 0)),
            pl.BlockSpec((NBLK, 1), lambda i: (i, 0)),
        ],
        out_shape=[
            jax.ShapeDtypeStruct((NP, 1), F32),
            jax.ShapeDtypeStruct((NP, 1), F32),
            jax.ShapeDtypeStruct((NP, 1), F32),
        ],
    )(deg2)


@functools.partial(
    pl.kernel,
    mesh=_MESH,
    compiler_params=pltpu.CompilerParams(needs_layout_passes=False),
    out_type=jax.ShapeDtypeStruct((2 * NP * 48,), F32),
    scratch_types=[
        pltpu.VMEM_SHARED((NP * 48,), F32),
        pltpu.VMEM((3840,), F32),
        pltpu.VMEM((NP,), F32),
        pltpu.VMEM((TPR2, ECH), I32),
        pltpu.VMEM((TPR2, ECH), I32),
        pltpu.VMEM((TPR2, ECH), I32),
        pltpu.VMEM((TPR2, ECH), I32),
        pltpu.VMEM((TPR2, ECH), I32),
        pltpu.VMEM((128,), F32),
        pltpu.VMEM((8, ECH), I32),
    ],
)
def _sc_tbins(row2d, col2d, ea0, ea1, ea2, dis_h,
              t_o, t_sp, zb, disv, rowt, colt, e0v, e1v, e2v, valb, idxb):
    c = lax.axis_index("c")
    s = lax.axis_index("s")

    def _fill(i, _):
        zb[pl.ds(i * 16, 16)] = _zero16()
        return 0
    lax.fori_loop(0, 240, _fill, 0)

    def _zt(i, _):
        pltpu.sync_copy(zb, t_sp.at[pl.ds(s * 30720 + i * 3840, 3840)])
        return 0
    lax.fori_loop(0, 8, _zt, 0)
    plsc.subcore_barrier()

    pltpu.sync_copy(dis_h, disv)

    base = c * (EROWS // 2) + s * TPR2
    pltpu.sync_copy(row2d.at[pl.ds(base, TPR2)], rowt)
    pltpu.sync_copy(col2d.at[pl.ds(base, TPR2)], colt)
    pltpu.sync_copy(ea0.at[pl.ds(base, TPR2)], e0v)
    pltpu.sync_copy(ea1.at[pl.ds(base, TPR2)], e1v)
    pltpu.sync_copy(ea2.at[pl.ds(base, TPR2)], e2v)

    def _tchunk(j, _):
        def _lane(k, _):
            sl = pl.ds(k * 16, 16)
            ridx = rowt.at[j][sl]
            valb[sl] = plsc.load_gather(disv, [ridx])
            cv = colt.at[j][sl] * 48
            idxb[0, sl] = cv + e0v.at[j][sl]
            idxb[1, sl] = cv + 16 + e1v.at[j][sl]
            idxb[2, sl] = cv + 32 + e2v.at[j][sl]
            return 0
        lax.fori_loop(0, 8, _lane, 0)
        pltpu.sync_copy(valb, t_sp.at[idxb.at[0]], add=True)
        pltpu.sync_copy(valb, t_sp.at[idxb.at[1]], add=True)
        pltpu.sync_copy(valb, t_sp.at[idxb.at[2]], add=True)
        return 0
    lax.fori_loop(0, TPR2, _tchunk, 0)
    plsc.subcore_barrier()

    pltpu.sync_copy(t_sp.at[pl.ds(s * 30720, 30720)],
                    t_o.at[pl.ds(c * (NP * 48) + s * 30720, 30720)])


SCH = 64
SROWS = EP // SCH
SPR = SROWS // 16


@functools.partial(
    pl.kernel,
    mesh=_MESH,
    compiler_params=pltpu.CompilerParams(needs_layout_passes=False),
    out_type=jax.ShapeDtypeStruct((2 * NP, 128), F32),
    scratch_types=[
        pltpu.VMEM_SHARED((NP, 128), F32),
        pltpu.VMEM((SPR // 2, SCH), I32),
        pltpu.VMEM((SPR // 2, SCH), I32),
        pltpu.VMEM((SCH, 128), F32),
        pltpu.VMEM((SCH, 128), F32),
        pltpu.VMEM((SCH, 128), F32),
        pltpu.SemaphoreType.DMA,
        pltpu.SemaphoreType.DMA,
        pltpu.SemaphoreType.DMA,
        pltpu.SemaphoreType.DMA,
        pltpu.SemaphoreType.DMA,
        pltpu.SemaphoreType.DMA,
    ],
)
def _sc_spmm(hs2, row2d, col2d, g_o, acc, rowv, colv,
             db0, db1, db2, gs0, gs1, gs2, ss0, ss1, ss2):
    dbufs = (db0, db1, db2)
    gsems = (gs0, gs1, gs2)
    ssems = (ss0, ss1, ss2)
    c = lax.axis_index("c")
    s = lax.axis_index("s")
    hpr = SPR // 2

    def _zl(k, _):
        rr = db0.at[k]

        def _zi(i, _):
            rr[pl.ds(i * 16, 16)] = _zero16()
            return 0
        lax.fori_loop(0, 8, _zi, 0)
        return 0
    lax.fori_loop(0, SCH, _zl, 0)

    def _zacc(i, _):
        pltpu.sync_copy(db0, acc.at[pl.ds(s * 640 + i * SCH, SCH)])
        return 0
    lax.fori_loop(0, 640 // SCH, _zacc, 0)
    plsc.subcore_barrier()

    off = c * NP
    for h in range(2):
        base = s * SPR + h * hpr
        pltpu.sync_copy(row2d.at[pl.ds(base, hpr)], rowv)
        pltpu.sync_copy(col2d.at[pl.ds(base, hpr)], colv)

        def _addoff(j, _):
            rr = rowv.at[j]

            def _al(k, _):
                sl = pl.ds(k * 16, 16)
                rr[sl] = rr[sl] + off
                return 0
            lax.fori_loop(0, SCH // 16, _al, 0)
            return 0
        lax.fori_loop(0, hpr, _addoff, 0)

        def _steps(jj, _):
            for b in range(3):
                t = 3 * jj + b
                b2 = (b + 2) % 3

                @pl.when(t < hpr)
                def _():
                    @pl.when(t >= 3)
                    def _():
                        pltpu.make_async_copy(
                            dbufs[b], acc.at[colv.at[t]], ssems[b]).wait()
                    pltpu.async_copy(hs2.at[rowv.at[t]], dbufs[b], gsems[b])

                @pl.when((t >= 1) & (t < hpr + 1))
                def _():
                    pltpu.make_async_copy(
                        hs2.at[rowv.at[jnp.maximum(t - 1, 0)]],
                        dbufs[b2], gsems[b2]).wait()
                    pltpu.async_copy(dbufs[b2],
                                     acc.at[colv.at[jnp.maximum(t - 1, 0)]],
                                     ssems[b2], add=True)
            return 0
        lax.fori_loop(0, (hpr + 1 + 2) // 3, _steps, 0)

        for b in range(3):
            pltpu.make_async_copy(dbufs[b], acc.at[colv.at[hpr - 3 + b]],
                                  ssems[b]).wait()
    plsc.subcore_barrier()

    pltpu.sync_copy(acc.at[pl.ds(s * 640, 640)],
                    g_o.at[pl.ds(c * NP + s * 640, 640)])


def _a0_body(x_ref, af_ref, wt_ref, dis_ref, o_ref):
    xb = x_ref[...]
    h0 = jnp.zeros((NBLK, EMB), F32)
    for f in range(9):
        oh = (xb[:, f][:, None]
              == lax.broadcasted_iota(I32, (1, 64), 1)).astype(F32)
        h0 = h0 + jnp.dot(oh, af_ref[pl.ds(f * 64, 64), :],
                          preferred_element_type=F32)
    hl = jnp.maximum(jnp.dot(h0, wt_ref[...], preferred_element_type=F32), 0.0)
    hs = hl * dis_ref[...]
    o_ref[0] = hs[:, :128]
    o_ref[1] = hs[:, 128:]


def _atom_layer0(x_p, atom_flat, w0t, dis):
    return pl.pallas_call(
        _a0_body,
        grid=(NGRID,),
        in_specs=[
            pl.BlockSpec((NBLK, 9), lambda i: (i, 0)),
            pl.BlockSpec((576, EMB), lambda i: (0, 0)),
            pl.BlockSpec((EMB, EMB), lambda i: (0, 0)),
            pl.BlockSpec((NBLK, 1), lambda i: (i, 0)),
        ],
        out_specs=pl.BlockSpec((2, NBLK, 128), lambda i: (0, i, 0)),
        out_shape=jax.ShapeDtypeStruct((2, NP, 128), F32),
    )(x_p, atom_flat, w0t, dis)


def _lt_phase1(i, g_ref, t_ref, bf_ref, hs_ref, dis1_ref, sdeg_ref,
               dinv_ref, root_ref, pre_sc, sacc):
    G = jnp.concatenate([g_ref[0], g_ref[1]], axis=1)
    HS = jnp.concatenate([hs_ref[0], hs_ref[1]], axis=1)
    Tb = t_ref[0] + t_ref[1]
    hl = HS * sdeg_ref[...]
    pre = (dis1_ref[...]
           * (G + jnp.dot(Tb, bf_ref[...], preferred_element_type=F32))
           + (hl + root_ref[...]) * dinv_ref[...])
    pre_sc[pl.ds(i * NBLK, NBLK), :] = pre
    gid = i * NBLK + lax.broadcasted_iota(I32, (NBLK, 1), 0)
    pm = jnp.where(gid < N, pre, 0.0)

    @pl.when(i == 0)
    def _():
        sacc[...] = jnp.zeros((8, EMB), F32)

    sacc[0:1, :] = sacc[0:1, :] + jnp.sum(pm, axis=0, keepdims=True)
    sacc[1:2, :] = sacc[1:2, :] + jnp.sum(pm * pm, axis=0, keepdims=True)


def _bn_from(sacc):
    mean = sacc[0:1, :] * (1.0 / N)
    var = sacc[1:2, :] * (1.0 / N) - mean * mean
    return mean, lax.rsqrt(var + 1e-5)


def _lt_body(g_ref, t_ref, bf_ref, hs_ref, dis1_ref, sdeg_ref, dinv_ref,
             root_ref, gm_ref, bt_ref, wt_ref, dis2_ref, o_ref,
             pre_sc, sacc):
    i = pl.program_id(0)

    @pl.when(i < NGRID)
    def _():
        _lt_phase1(i, g_ref, t_ref, bf_ref, hs_ref, dis1_ref, sdeg_ref,
                   dinv_ref, root_ref, pre_sc, sacc)

    @pl.when(i >= NGRID)
    def _():
        ii = i - NGRID
        pre = pre_sc[pl.ds(ii * NBLK, NBLK), :]
        mean, inv = _bn_from(sacc)
        h = (pre - mean) * inv * gm_ref[...] + bt_ref[...]
        h = jnp.maximum(h, 0.0)
        hl = jnp.maximum(jnp.dot(h, wt_ref[...], preferred_element_type=F32),
                         0.0)
        hs = hl * dis2_ref[...]
        o_ref[0] = hs[:, :128]
        o_ref[1] = hs[:, 128:]


def _c1(i):
    return (0, jnp.minimum(i, NGRID - 1), 0)


def _r1(i):
    return (jnp.minimum(i, NGRID - 1), 0)


_P1_SPECS = [
    pl.BlockSpec((2, NBLK, 128), _c1),
    pl.BlockSpec((2, NBLK, 48), _c1),
    pl.BlockSpec((48, EMB), lambda i: (0, 0)),
    pl.BlockSpec((2, NBLK, 128), _c1),
    pl.BlockSpec((NBLK, 1), _r1),
    pl.BlockSpec((NBLK, 1), _r1),
    pl.BlockSpec((NBLK, 1), _r1),
    pl.BlockSpec((1, EMB), lambda i: (0, 0)),
]


def _layer_tc(g3, t3, bondflat, hs3, dis, sdeg, dinv, root_l,
              gamma_l, beta_l, wt):
    return pl.pallas_call(
        _lt_body,
        grid=(2 * NGRID,),
        in_specs=_P1_SPECS + [
            pl.BlockSpec((1, EMB), lambda i: (0, 0)),
            pl.BlockSpec((1, EMB), lambda i: (0, 0)),
            pl.BlockSpec((EMB, EMB), lambda i: (0, 0)),
            pl.BlockSpec((NBLK, 1), lambda i: (jnp.maximum(i - NGRID, 0), 0)),
        ],
        out_specs=pl.BlockSpec((2, NBLK, 128),
                               lambda i: (0, jnp.maximum(i - NGRID, 0), 0)),
        out_shape=jax.ShapeDtypeStruct((2, NP, 128), F32),
        scratch_shapes=[pltpu.VMEM((NP, EMB), F32), pltpu.VMEM((8, EMB), F32)],
    )(g3, t3, bondflat, hs3, dis, sdeg, dinv, root_l, gamma_l, beta_l,
      wt, dis)


FBLK = 1000


def _ltf_body(g_ref, t_ref, bf_ref, hs_ref, dis1_ref, sdeg_ref, dinv_ref,
              root_ref, gm_ref, bt_ref, o_ref, pre_sc, sacc):
    i = pl.program_id(0)

    @pl.when(i < NGRID)
    def _():
        _lt_phase1(i, g_ref, t_ref, bf_ref, hs_ref, dis1_ref, sdeg_ref,
                   dinv_ref, root_ref, pre_sc, sacc)

    @pl.when(i >= NGRID)
    def _():
        ii = i - NGRID
        pre = pre_sc[pl.ds(ii * FBLK, FBLK), :]
        mean, inv = _bn_from(sacc)
        o_ref[...] = (pre - mean) * inv * gm_ref[...] + bt_ref[...]


def _layer_tc_final(g3, t3, bondflat, hs3, dis, sdeg, dinv, root_l,
                    gamma_l, beta_l):
    return pl.pallas_call(
        _ltf_body,
        grid=(2 * NGRID,),
        in_specs=_P1_SPECS + [
            pl.BlockSpec((1, EMB), lambda i: (0, 0)),
            pl.BlockSpec((1, EMB), lambda i: (0, 0)),
        ],
        out_specs=pl.BlockSpec((FBLK, EMB),
                               lambda i: (jnp.maximum(i - NGRID, 0), 0)),
        out_shape=jax.ShapeDtypeStruct((N, EMB), F32),
        scratch_shapes=[pltpu.VMEM((NP, EMB), F32), pltpu.VMEM((8, EMB), F32)],
    )(g3, t3, bondflat, hs3, dis, sdeg, dinv, root_l, gamma_l, beta_l)


def kernel(x, edge_index, edge_attr, atom_tab, W, root, bond, gamma, beta):
    row = edge_index[0].astype(I32)
    col = edge_index[1].astype(I32)
    pad_e = EP - E
    pad_ids = (N + (jnp.arange(pad_e, dtype=I32) % (NP - N))).astype(I32)
    row2d = jnp.concatenate([row, pad_ids]).reshape(EROWS, ECH)
    col2d = jnp.concatenate([col, pad_ids]).reshape(EROWS, ECH)
    eap = jnp.concatenate(
        [edge_attr.astype(I32), jnp.zeros((pad_e, 3), I32)], axis=0)
    ea0 = eap[:, 0].reshape(EROWS, ECH)
    ea1 = eap[:, 1].reshape(EROWS, ECH)
    ea2 = eap[:, 2].reshape(EROWS, ECH)
    x_p = jnp.concatenate(
        [x.astype(I32), jnp.zeros((NP - N, x.shape[1]), I32)], axis=0)

    atom_flat = atom_tab.reshape(576, EMB)
    wts = [W[l].T for l in range(3)]
    bfs = [bond[l].reshape(48, EMB) for l in range(3)]

    deg2 = _sc_deg(row2d).reshape(2, NP, 1)
    dis, sdeg, dinv = _deg_norm(deg2)
    t_flat = _sc_tbins(row2d, col2d, ea0, ea1, ea2, dis.reshape(NP))
    t3 = t_flat.reshape(2, NP, 48)

    hs3 = _atom_layer0(x_p, atom_flat, wts[0], dis)
    out = None
    for l in range(3):
        g2 = _sc_spmm(hs3.reshape(2 * NP, 128),
                      row2d.reshape(SROWS, SCH), col2d.reshape(SROWS, SCH))
        g3 = g2.reshape(2, NP, 128)
        if l < 2:
            hs3 = _layer_tc(g3, t3, bfs[l], hs3, dis, sdeg, dinv,
                            root[l][None, :], gamma[l][None, :],
                            beta[l][None, :], wts[l + 1])
        else:
            out = _layer_tc_final(g3, t3, bfs[l], hs3, dis, sdeg, dinv,
                                  root[l][None, :], gamma[l][None, :],
                                  beta[l][None, :])
    return out


# speedup vs baseline: 17.4605x; 1.0492x over previous
import functools

import jax
import jax.numpy as jnp
from jax import lax
from jax.experimental import pallas as pl
from jax.experimental.pallas import tpu as pltpu
from jax.experimental.pallas import tpu_sc as plsc

F32 = jnp.float32
I32 = jnp.int32

N = 10000
NP = 10240
E = 160000
EP = 163840
EMB = 256
NBLK = 1024
NGRID = NP // NBLK
ECH = 128
EROWS = EP // ECH
TPR = EROWS // 16
TPR2 = EROWS // 32

_MESH = plsc.VectorSubcoreMesh(core_axis_name="c", subcore_axis_name="s")


def _zero16():
    return jnp.zeros((16,), F32)


DBASE = EROWS // 32


@functools.partial(
    pl.kernel,
    mesh=_MESH,
    compiler_params=pltpu.CompilerParams(needs_layout_passes=False),
    out_type=jax.ShapeDtypeStruct((2 * NP,), F32),
    scratch_types=[
        pltpu.VMEM_SHARED((NP,), F32),
        pltpu.VMEM((640,), F32),
        pltpu.VMEM((128,), F32),
        pltpu.VMEM((DBASE, ECH), I32),
    ],
)
def _sc_deg(row2d, deg_o, deg_sp, zb, ones, rowv):
    c = lax.axis_index("c")
    s = lax.axis_index("s")
    w = s * 2 + c

    def _fill(i, _):
        zb[pl.ds(i * 16, 16)] = _zero16()
        return 0
    lax.fori_loop(0, 40, _fill, 0)

    def _fill1(i, _):
        ones[pl.ds(i * 16, 16)] = jnp.ones((16,), F32)
        return 0
    lax.fori_loop(0, 8, _fill1, 0)

    pltpu.sync_copy(zb, deg_sp.at[pl.ds(s * 640, 640)])
    plsc.subcore_barrier()

    base = w * DBASE
    pltpu.sync_copy(row2d.at[pl.ds(base, DBASE)], rowv)

    def _deg(j, _):
        pltpu.sync_copy(ones, deg_sp.at[rowv.at[j]], add=True)
        return 0
    lax.fori_loop(0, DBASE, _deg, 0)
    plsc.subcore_barrier()

    pltpu.sync_copy(deg_sp.at[pl.ds(s * 640, 640)],
                    deg_o.at[pl.ds(c * NP + s * 640, 640)])


def _dn_body(deg_ref, dis_ref, sdeg_ref, dinv_ref):
    degp = deg_ref[0] + deg_ref[1] + 1.0
    y = lax.rsqrt(degp)
    dis_ref[...] = y
    sdeg_ref[...] = degp * y
    dinv_ref[...] = y * y


def _deg_norm(deg2):
    return pl.pallas_call(
        _dn_body,
        grid=(1,),
        in_specs=[pl.BlockSpec((2, NP // 128, 128), lambda i: (0, 0, 0))],
        out_specs=[
            pl.BlockSpec((NP // 128, 128), lambda i: (0, 0)),
            pl.BlockSpec((NP // 128, 128), lambda i: (0, 0)),
            pl.BlockSpec((NP // 128, 128), lambda i: (0, 0)),
        ],
        out_shape=[
            jax.ShapeDtypeStruct((NP // 128, 128), F32),
            jax.ShapeDtypeStruct((NP // 128, 128), F32),
            jax.ShapeDtypeStruct((NP // 128, 128), F32),
        ],
    )(deg2)


@functools.partial(
    pl.kernel,
    mesh=_MESH,
    compiler_params=pltpu.CompilerParams(needs_layout_passes=False),
    out_type=jax.ShapeDtypeStruct((2 * NP * 48,), F32),
    scratch_types=[
        pltpu.VMEM_SHARED((NP * 48,), F32),
        pltpu.VMEM((3840,), F32),
        pltpu.VMEM((NP,), F32),
        pltpu.VMEM((TPR2, ECH), I32),
        pltpu.VMEM((TPR2, ECH), I32),
        pltpu.VMEM((TPR2, ECH), I32),
        pltpu.VMEM((TPR2, ECH), I32),
        pltpu.VMEM((TPR2, ECH), I32),
        pltpu.VMEM((128,), F32),
        pltpu.VMEM((8, ECH), I32),
        pltpu.VMEM((128,), F32),
        pltpu.VMEM((8, ECH), I32),
        pltpu.SemaphoreType.DMA,
        pltpu.SemaphoreType.DMA,
    ],
)
def _sc_tbins(row2d, col2d, ea0, ea1, ea2, dis_h,
              t_o, t_sp, zb, disv, rowt, colt, e0v, e1v, e2v,
              vb0, ib0, vb1, ib1, ts0, ts1):
    valbs = (vb0, vb1)
    idxbs = (ib0, ib1)
    tsems = (ts0, ts1)
    c = lax.axis_index("c")
    s = lax.axis_index("s")

    def _fill(i, _):
        zb[pl.ds(i * 16, 16)] = _zero16()
        return 0
    lax.fori_loop(0, 240, _fill, 0)

    def _zt(i, _):
        pltpu.sync_copy(zb, t_sp.at[pl.ds(s * 30720 + i * 3840, 3840)])
        return 0
    lax.fori_loop(0, 8, _zt, 0)
    plsc.subcore_barrier()

    pltpu.sync_copy(dis_h, disv)

    base = c * (EROWS // 2) + s * TPR2
    pltpu.sync_copy(row2d.at[pl.ds(base, TPR2)], rowt)
    pltpu.sync_copy(col2d.at[pl.ds(base, TPR2)], colt)
    pltpu.sync_copy(ea0.at[pl.ds(base, TPR2)], e0v)
    pltpu.sync_copy(ea1.at[pl.ds(base, TPR2)], e1v)
    pltpu.sync_copy(ea2.at[pl.ds(base, TPR2)], e2v)

    def _tchunk(jj, _):
        for b in range(2):
            j = 2 * jj + b
            valb, idxb, tsem = valbs[b], idxbs[b], tsems[b]

            @pl.when(j >= 2)
            def _():
                for f in range(3):
                    pltpu.make_async_copy(valb, t_sp.at[idxb.at[f]],
                                          tsem).wait()

            def _lane(k, _):
                sl = pl.ds(k * 16, 16)
                ridx = rowt.at[j][sl]
                valb[sl] = plsc.load_gather(disv, [ridx])
                cv = colt.at[j][sl] * 48
                idxb[0, sl] = cv + e0v.at[j][sl]
                idxb[1, sl] = cv + 16 + e1v.at[j][sl]
                idxb[2, sl] = cv + 32 + e2v.at[j][sl]
                return 0
            lax.fori_loop(0, 8, _lane, 0)
            for f in range(3):
                pltpu.async_copy(valb, t_sp.at[idxb.at[f]], tsem)
        return 0
    lax.fori_loop(0, TPR2 // 2, _tchunk, 0)
    for b in range(2):
        for f in range(3):
            pltpu.make_async_copy(valbs[b], t_sp.at[idxbs[b].at[f]],
                                  tsems[b]).wait()
    plsc.subcore_barrier()

    pltpu.sync_copy(t_sp.at[pl.ds(s * 30720, 30720)],
                    t_o.at[pl.ds(c * (NP * 48) + s * 30720, 30720)])


SCH = 64
SROWS = EP // SCH
SPR = SROWS // 16


@functools.partial(
    pl.kernel,
    mesh=_MESH,
    compiler_params=pltpu.CompilerParams(needs_layout_passes=False),
    out_type=jax.ShapeDtypeStruct((2 * NP, 128), F32),
    scratch_types=[
        pltpu.VMEM_SHARED((NP, 128), F32),
        pltpu.VMEM((SPR // 2, SCH), I32),
        pltpu.VMEM((SPR // 2, SCH), I32),
        pltpu.VMEM((SCH, 128), F32),
        pltpu.VMEM((SCH, 128), F32),
        pltpu.VMEM((SCH, 128), F32),
        pltpu.SemaphoreType.DMA,
        pltpu.SemaphoreType.DMA,
        pltpu.SemaphoreType.DMA,
        pltpu.SemaphoreType.DMA,
        pltpu.SemaphoreType.DMA,
        pltpu.SemaphoreType.DMA,
    ],
)
def _sc_spmm(hs2, row2d, col2d, g_o, acc, rowv, colv,
             db0, db1, db2, gs0, gs1, gs2, ss0, ss1, ss2):
    dbufs = (db0, db1, db2)
    gsems = (gs0, gs1, gs2)
    ssems = (ss0, ss1, ss2)
    c = lax.axis_index("c")
    s = lax.axis_index("s")
    hpr = SPR // 2

    def _zl(k, _):
        rr = db0.at[k]

        def _zi(i, _):
            rr[pl.ds(i * 16, 16)] = _zero16()
            return 0
        lax.fori_loop(0, 8, _zi, 0)
        return 0
    lax.fori_loop(0, SCH, _zl, 0)

    def _zacc(i, _):
        pltpu.sync_copy(db0, acc.at[pl.ds(s * 640 + i * SCH, SCH)])
        return 0
    lax.fori_loop(0, 640 // SCH, _zacc, 0)
    plsc.subcore_barrier()

    off = c * NP
    for h in range(2):
        base = s * SPR + h * hpr
        pltpu.sync_copy(row2d.at[pl.ds(base, hpr)], rowv)
        pltpu.sync_copy(col2d.at[pl.ds(base, hpr)], colv)

        def _addoff(j, _):
            rr = rowv.at[j]

            def _al(k, _):
                sl = pl.ds(k * 16, 16)
                rr[sl] = rr[sl] + off
                return 0
            lax.fori_loop(0, SCH // 16, _al, 0)
            return 0
        lax.fori_loop(0, hpr, _addoff, 0)

        def _steps(jj, _):
            for b in range(3):
                t = 3 * jj + b
                b2 = (b + 2) % 3

                @pl.when(t < hpr)
                def _():
                    @pl.when(t >= 3)
                    def _():
                        pltpu.make_async_copy(
                            dbufs[b], acc.at[colv.at[t]], ssems[b]).wait()
                    pltpu.async_copy(hs2.at[rowv.at[t]], dbufs[b], gsems[b])

                @pl.when((t >= 1) & (t < hpr + 1))
                def _():
                    pltpu.make_async_copy(
                        hs2.at[rowv.at[jnp.maximum(t - 1, 0)]],
                        dbufs[b2], gsems[b2]).wait()
                    pltpu.async_copy(dbufs[b2],
                                     acc.at[colv.at[jnp.maximum(t - 1, 0)]],
                                     ssems[b2], add=True)
            return 0
        lax.fori_loop(0, (hpr + 1 + 2) // 3, _steps, 0)

        for b in range(3):
            pltpu.make_async_copy(dbufs[b], acc.at[colv.at[hpr - 3 + b]],
                                  ssems[b]).wait()
    plsc.subcore_barrier()

    pltpu.sync_copy(acc.at[pl.ds(s * 640, 640)],
                    g_o.at[pl.ds(c * NP + s * 640, 640)])


def _a0_body(x_ref, af_ref, wt_ref, dis_ref, o_ref):
    xb = x_ref[...]
    h0 = jnp.zeros((NBLK, EMB), F32)
    for f in range(9):
        oh = (xb[:, f][:, None]
              == lax.broadcasted_iota(I32, (1, 64), 1)).astype(F32)
        h0 = h0 + jnp.dot(oh, af_ref[pl.ds(f * 64, 64), :],
                          preferred_element_type=F32)
    hl = jnp.maximum(jnp.dot(h0, wt_ref[...], preferred_element_type=F32), 0.0)
    hs = hl * dis_ref[...]
    o_ref[0] = hs[:, :128]
    o_ref[1] = hs[:, 128:]


def _atom_layer0(x_p, atom_flat, w0t, dis):
    return pl.pallas_call(
        _a0_body,
        grid=(NGRID,),
        in_specs=[
            pl.BlockSpec((NBLK, 9), lambda i: (i, 0)),
            pl.BlockSpec((576, EMB), lambda i: (0, 0)),
            pl.BlockSpec((EMB, EMB), lambda i: (0, 0)),
            pl.BlockSpec((NBLK, 1), lambda i: (i, 0)),
        ],
        out_specs=pl.BlockSpec((2, NBLK, 128), lambda i: (0, i, 0)),
        out_shape=jax.ShapeDtypeStruct((2, NP, 128), F32),
    )(x_p, atom_flat, w0t, dis)


def _lt_phase1(i, g_ref, t_ref, bf_ref, hs_ref, dis1_ref, sdeg_ref,
               dinv_ref, root_ref, pre_sc, sacc):
    G = jnp.concatenate([g_ref[0], g_ref[1]], axis=1)
    HS = jnp.concatenate([hs_ref[0], hs_ref[1]], axis=1)
    Tb = t_ref[0] + t_ref[1]
    hl = HS * sdeg_ref[...]
    pre = (dis1_ref[...]
           * (G + jnp.dot(Tb, bf_ref[...], preferred_element_type=F32))
           + (hl + root_ref[...]) * dinv_ref[...])
    pre_sc[pl.ds(i * NBLK, NBLK), :] = pre
    gid = i * NBLK + lax.broadcasted_iota(I32, (NBLK, 1), 0)
    pm = jnp.where(gid < N, pre, 0.0)

    @pl.when(i == 0)
    def _():
        sacc[...] = jnp.zeros((8, EMB), F32)

    sacc[0:1, :] = sacc[0:1, :] + jnp.sum(pm, axis=0, keepdims=True)
    sacc[1:2, :] = sacc[1:2, :] + jnp.sum(pm * pm, axis=0, keepdims=True)


def _bn_from(sacc):
    mean = sacc[0:1, :] * (1.0 / N)
    var = sacc[1:2, :] * (1.0 / N) - mean * mean
    return mean, lax.rsqrt(var + 1e-5)


def _lt_body(g_ref, t_ref, bf_ref, hs_ref, dis1_ref, sdeg_ref, dinv_ref,
             root_ref, gm_ref, bt_ref, wt_ref, dis2_ref, o_ref,
             pre_sc, sacc):
    i = pl.program_id(0)

    @pl.when(i < NGRID)
    def _():
        _lt_phase1(i, g_ref, t_ref, bf_ref, hs_ref, dis1_ref, sdeg_ref,
                   dinv_ref, root_ref, pre_sc, sacc)

    @pl.when(i >= NGRID)
    def _():
        ii = i - NGRID
        pre = pre_sc[pl.ds(ii * NBLK, NBLK), :]
        mean, inv = _bn_from(sacc)
        h = (pre - mean) * inv * gm_ref[...] + bt_ref[...]
        h = jnp.maximum(h, 0.0)
        hl = jnp.maximum(jnp.dot(h, wt_ref[...], preferred_element_type=F32),
                         0.0)
        hs = hl * dis2_ref[...]
        o_ref[0] = hs[:, :128]
        o_ref[1] = hs[:, 128:]


def _c1(i):
    return (0, jnp.minimum(i, NGRID - 1), 0)


def _r1(i):
    return (jnp.minimum(i, NGRID - 1), 0)


_P1_SPECS = [
    pl.BlockSpec((2, NBLK, 128), _c1),
    pl.BlockSpec((2, NBLK, 48), _c1),
    pl.BlockSpec((48, EMB), lambda i: (0, 0)),
    pl.BlockSpec((2, NBLK, 128), _c1),
    pl.BlockSpec((NBLK, 1), _r1),
    pl.BlockSpec((NBLK, 1), _r1),
    pl.BlockSpec((NBLK, 1), _r1),
    pl.BlockSpec((1, EMB), lambda i: (0, 0)),
]


def _layer_tc(g3, t3, bondflat, hs3, dis, sdeg, dinv, root_l,
              gamma_l, beta_l, wt):
    return pl.pallas_call(
        _lt_body,
        grid=(2 * NGRID,),
        in_specs=_P1_SPECS + [
            pl.BlockSpec((1, EMB), lambda i: (0, 0)),
            pl.BlockSpec((1, EMB), lambda i: (0, 0)),
            pl.BlockSpec((EMB, EMB), lambda i: (0, 0)),
            pl.BlockSpec((NBLK, 1), lambda i: (jnp.maximum(i - NGRID, 0), 0)),
        ],
        out_specs=pl.BlockSpec((2, NBLK, 128),
                               lambda i: (0, jnp.maximum(i - NGRID, 0), 0)),
        out_shape=jax.ShapeDtypeStruct((2, NP, 128), F32),
        scratch_shapes=[pltpu.VMEM((NP, EMB), F32), pltpu.VMEM((8, EMB), F32)],
    )(g3, t3, bondflat, hs3, dis, sdeg, dinv, root_l, gamma_l, beta_l,
      wt, dis)


FBLK = 1000


def _ltf_body(g_ref, t_ref, bf_ref, hs_ref, dis1_ref, sdeg_ref, dinv_ref,
              root_ref, gm_ref, bt_ref, o_ref, pre_sc, sacc):
    i = pl.program_id(0)

    @pl.when(i < NGRID)
    def _():
        _lt_phase1(i, g_ref, t_ref, bf_ref, hs_ref, dis1_ref, sdeg_ref,
                   dinv_ref, root_ref, pre_sc, sacc)

    @pl.when(i >= NGRID)
    def _():
        ii = i - NGRID
        pre = pre_sc[pl.ds(ii * FBLK, FBLK), :]
        mean, inv = _bn_from(sacc)
        o_ref[...] = (pre - mean) * inv * gm_ref[...] + bt_ref[...]


def _layer_tc_final(g3, t3, bondflat, hs3, dis, sdeg, dinv, root_l,
                    gamma_l, beta_l):
    return pl.pallas_call(
        _ltf_body,
        grid=(2 * NGRID,),
        in_specs=_P1_SPECS + [
            pl.BlockSpec((1, EMB), lambda i: (0, 0)),
            pl.BlockSpec((1, EMB), lambda i: (0, 0)),
        ],
        out_specs=pl.BlockSpec((FBLK, EMB),
                               lambda i: (jnp.maximum(i - NGRID, 0), 0)),
        out_shape=jax.ShapeDtypeStruct((N, EMB), F32),
        scratch_shapes=[pltpu.VMEM((NP, EMB), F32), pltpu.VMEM((8, EMB), F32)],
    )(g3, t3, bondflat, hs3, dis, sdeg, dinv, root_l, gamma_l, beta_l)


def kernel(x, edge_index, edge_attr, atom_tab, W, root, bond, gamma, beta):
    row = edge_index[0].astype(I32)
    col = edge_index[1].astype(I32)
    pad_e = EP - E
    pad_ids = (N + (jnp.arange(pad_e, dtype=I32) % (NP - N))).astype(I32)
    row2d = jnp.concatenate([row, pad_ids]).reshape(EROWS, ECH)
    col2d = jnp.concatenate([col, pad_ids]).reshape(EROWS, ECH)
    eap = jnp.concatenate(
        [edge_attr.astype(I32), jnp.zeros((pad_e, 3), I32)], axis=0)
    ea0 = eap[:, 0].reshape(EROWS, ECH)
    ea1 = eap[:, 1].reshape(EROWS, ECH)
    ea2 = eap[:, 2].reshape(EROWS, ECH)
    x_p = jnp.concatenate(
        [x.astype(I32), jnp.zeros((NP - N, x.shape[1]), I32)], axis=0)

    atom_flat = atom_tab.reshape(576, EMB)
    wts = [W[l].T for l in range(3)]
    bfs = [bond[l].reshape(48, EMB) for l in range(3)]

    deg2 = _sc_deg(row2d).reshape(2, NP // 128, 128)
    dis, sdeg, dinv = _deg_norm(deg2)
    dis = dis.reshape(NP, 1)
    sdeg = sdeg.reshape(NP, 1)
    dinv = dinv.reshape(NP, 1)
    t_flat = _sc_tbins(row2d, col2d, ea0, ea1, ea2, dis.reshape(NP))
    t3 = t_flat.reshape(2, NP, 48)

    hs3 = _atom_layer0(x_p, atom_flat, wts[0], dis)
    out = None
    for l in range(3):
        g2 = _sc_spmm(hs3.reshape(2 * NP, 128),
                      row2d.reshape(SROWS, SCH), col2d.reshape(SROWS, SCH))
        g3 = g2.reshape(2, NP, 128)
        if l < 2:
            hs3 = _layer_tc(g3, t3, bfs[l], hs3, dis, sdeg, dinv,
                            root[l][None, :], gamma[l][None, :],
                            beta[l][None, :], wts[l + 1])
        else:
            out = _layer_tc_final(g3, t3, bfs[l], hs3, dis, sdeg, dinv,
                                  root[l][None, :], gamma[l][None, :],
                                  beta[l][None, :])
    return out


# speedup vs baseline: 17.4614x; 1.0001x over previous
import functools

import jax
import jax.numpy as jnp
from jax import lax
from jax.experimental import pallas as pl
from jax.experimental.pallas import tpu as pltpu
from jax.experimental.pallas import tpu_sc as plsc

F32 = jnp.float32
I32 = jnp.int32

N = 10000
NP = 10240
E = 160000
EP = 163840
EMB = 256
NBLK = 1024
NGRID = NP // NBLK
ECH = 128
EROWS = EP // ECH
TPR = EROWS // 16
TPR2 = EROWS // 32

_MESH = plsc.VectorSubcoreMesh(core_axis_name="c", subcore_axis_name="s")


def _zero16():
    return jnp.zeros((16,), F32)


DBASE = EROWS // 32


@functools.partial(
    pl.kernel,
    mesh=_MESH,
    compiler_params=pltpu.CompilerParams(needs_layout_passes=False),
    out_type=jax.ShapeDtypeStruct((2 * NP,), F32),
    scratch_types=[
        pltpu.VMEM_SHARED((NP,), F32),
        pltpu.VMEM((640,), F32),
        pltpu.VMEM((128,), F32),
        pltpu.VMEM((DBASE, ECH), I32),
    ],
)
def _sc_deg(row2d, deg_o, deg_sp, zb, ones, rowv):
    c = lax.axis_index("c")
    s = lax.axis_index("s")
    w = s * 2 + c

    def _fill(i, _):
        zb[pl.ds(i * 16, 16)] = _zero16()
        return 0
    lax.fori_loop(0, 40, _fill, 0)

    def _fill1(i, _):
        ones[pl.ds(i * 16, 16)] = jnp.ones((16,), F32)
        return 0
    lax.fori_loop(0, 8, _fill1, 0)

    pltpu.sync_copy(zb, deg_sp.at[pl.ds(s * 640, 640)])
    plsc.subcore_barrier()

    base = w * DBASE
    pltpu.sync_copy(row2d.at[pl.ds(base, DBASE)], rowv)

    def _deg(j, _):
        pltpu.sync_copy(ones, deg_sp.at[rowv.at[j]], add=True)
        return 0
    lax.fori_loop(0, DBASE, _deg, 0)
    plsc.subcore_barrier()

    pltpu.sync_copy(deg_sp.at[pl.ds(s * 640, 640)],
                    deg_o.at[pl.ds(c * NP + s * 640, 640)])


def _dn_body(deg_ref, dis_ref, sdeg_ref, dinv_ref):
    degp = deg_ref[0] + deg_ref[1] + 1.0
    y = lax.rsqrt(degp)
    dis_ref[...] = y
    sdeg_ref[...] = degp * y
    dinv_ref[...] = y * y


def _deg_norm(deg2):
    return pl.pallas_call(
        _dn_body,
        grid=(1,),
        in_specs=[pl.BlockSpec((2, NP // 128, 128), lambda i: (0, 0, 0))],
        out_specs=[
            pl.BlockSpec((NP // 128, 128), lambda i: (0, 0)),
            pl.BlockSpec((NP // 128, 128), lambda i: (0, 0)),
            pl.BlockSpec((NP // 128, 128), lambda i: (0, 0)),
        ],
        out_shape=[
            jax.ShapeDtypeStruct((NP // 128, 128), F32),
            jax.ShapeDtypeStruct((NP // 128, 128), F32),
            jax.ShapeDtypeStruct((NP // 128, 128), F32),
        ],
    )(deg2)


@functools.partial(
    pl.kernel,
    mesh=_MESH,
    compiler_params=pltpu.CompilerParams(needs_layout_passes=False),
    out_type=jax.ShapeDtypeStruct((2 * NP * 48,), F32),
    scratch_types=[
        pltpu.VMEM_SHARED((NP * 48,), F32),
        pltpu.VMEM((3840,), F32),
        pltpu.VMEM((NP,), F32),
        pltpu.VMEM((TPR2, ECH), I32),
        pltpu.VMEM((TPR2, ECH), I32),
        pltpu.VMEM((TPR2, ECH), I32),
        pltpu.VMEM((TPR2, ECH), I32),
        pltpu.VMEM((TPR2, ECH), I32),
        pltpu.VMEM((128,), F32),
        pltpu.VMEM((8, ECH), I32),
        pltpu.VMEM((128,), F32),
        pltpu.VMEM((8, ECH), I32),
        pltpu.SemaphoreType.DMA,
        pltpu.SemaphoreType.DMA,
    ],
)
def _sc_tbins(row2d, col2d, ea0, ea1, ea2, dis_h,
              t_o, t_sp, zb, disv, rowt, colt, e0v, e1v, e2v,
              vb0, ib0, vb1, ib1, ts0, ts1):
    valbs = (vb0, vb1)
    idxbs = (ib0, ib1)
    tsems = (ts0, ts1)
    c = lax.axis_index("c")
    s = lax.axis_index("s")

    def _fill(i, _):
        zb[pl.ds(i * 16, 16)] = _zero16()
        return 0
    lax.fori_loop(0, 240, _fill, 0)

    def _zt(i, _):
        pltpu.sync_copy(zb, t_sp.at[pl.ds(s * 30720 + i * 3840, 3840)])
        return 0
    lax.fori_loop(0, 8, _zt, 0)
    plsc.subcore_barrier()

    pltpu.sync_copy(dis_h, disv)

    base = c * (EROWS // 2) + s * TPR2
    pltpu.sync_copy(row2d.at[pl.ds(base, TPR2)], rowt)
    pltpu.sync_copy(col2d.at[pl.ds(base, TPR2)], colt)
    pltpu.sync_copy(ea0.at[pl.ds(base, TPR2)], e0v)
    pltpu.sync_copy(ea1.at[pl.ds(base, TPR2)], e1v)
    pltpu.sync_copy(ea2.at[pl.ds(base, TPR2)], e2v)

    def _tchunk(jj, _):
        for b in range(2):
            j = 2 * jj + b
            valb, idxb, tsem = valbs[b], idxbs[b], tsems[b]

            @pl.when(j >= 2)
            def _():
                for f in range(3):
                    pltpu.make_async_copy(valb, t_sp.at[idxb.at[f]],
                                          tsem).wait()

            def _lane(k, _):
                sl = pl.ds(k * 16, 16)
                ridx = rowt.at[j][sl]
                valb[sl] = plsc.load_gather(disv, [ridx])
                cv = colt.at[j][sl] * 48
                idxb[0, sl] = cv + e0v.at[j][sl]
                idxb[1, sl] = cv + 16 + e1v.at[j][sl]
                idxb[2, sl] = cv + 32 + e2v.at[j][sl]
                return 0
            lax.fori_loop(0, 8, _lane, 0)
            for f in range(3):
                pltpu.async_copy(valb, t_sp.at[idxb.at[f]], tsem, add=True)
        return 0
    lax.fori_loop(0, TPR2 // 2, _tchunk, 0)
    for b in range(2):
        for f in range(3):
            pltpu.make_async_copy(valbs[b], t_sp.at[idxbs[b].at[f]],
                                  tsems[b]).wait()
    plsc.subcore_barrier()

    pltpu.sync_copy(t_sp.at[pl.ds(s * 30720, 30720)],
                    t_o.at[pl.ds(c * (NP * 48) + s * 30720, 30720)])


SCH = 64
SROWS = EP // SCH
SPR = SROWS // 16


@functools.partial(
    pl.kernel,
    mesh=_MESH,
    compiler_params=pltpu.CompilerParams(needs_layout_passes=False),
    out_type=jax.ShapeDtypeStruct((2 * NP, 128), F32),
    scratch_types=[
        pltpu.VMEM_SHARED((NP, 128), F32),
        pltpu.VMEM((SPR // 2, SCH), I32),
        pltpu.VMEM((SPR // 2, SCH), I32),
        pltpu.VMEM((SCH, 128), F32),
        pltpu.VMEM((SCH, 128), F32),
        pltpu.VMEM((SCH, 128), F32),
        pltpu.SemaphoreType.DMA,
        pltpu.SemaphoreType.DMA,
        pltpu.SemaphoreType.DMA,
        pltpu.SemaphoreType.DMA,
        pltpu.SemaphoreType.DMA,
        pltpu.SemaphoreType.DMA,
    ],
)
def _sc_spmm(hs2, row2d, col2d, g_o, acc, rowv, colv,
             db0, db1, db2, gs0, gs1, gs2, ss0, ss1, ss2):
    dbufs = (db0, db1, db2)
    gsems = (gs0, gs1, gs2)
    ssems = (ss0, ss1, ss2)
    c = lax.axis_index("c")
    s = lax.axis_index("s")
    hpr = SPR // 2

    def _zl(k, _):
        rr = db0.at[k]

        def _zi(i, _):
            rr[pl.ds(i * 16, 16)] = _zero16()
            return 0
        lax.fori_loop(0, 8, _zi, 0)
        return 0
    lax.fori_loop(0, SCH, _zl, 0)

    def _zacc(i, _):
        pltpu.sync_copy(db0, acc.at[pl.ds(s * 640 + i * SCH, SCH)])
        return 0
    lax.fori_loop(0, 640 // SCH, _zacc, 0)
    plsc.subcore_barrier()

    off = c * NP
    for h in range(2):
        base = s * SPR + h * hpr
        pltpu.sync_copy(row2d.at[pl.ds(base, hpr)], rowv)
        pltpu.sync_copy(col2d.at[pl.ds(base, hpr)], colv)

        def _addoff(j, _):
            rr = rowv.at[j]

            def _al(k, _):
                sl = pl.ds(k * 16, 16)
                rr[sl] = rr[sl] + off
                return 0
            lax.fori_loop(0, SCH // 16, _al, 0)
            return 0
        lax.fori_loop(0, hpr, _addoff, 0)

        def _steps(jj, _):
            for b in range(3):
                t = 3 * jj + b
                b2 = (b + 2) % 3

                @pl.when(t < hpr)
                def _():
                    @pl.when(t >= 3)
                    def _():
                        pltpu.make_async_copy(
                            dbufs[b], acc.at[colv.at[t]], ssems[b]).wait()
                    pltpu.async_copy(hs2.at[rowv.at[t]], dbufs[b], gsems[b])

                @pl.when((t >= 1) & (t < hpr + 1))
                def _():
                    pltpu.make_async_copy(
                        hs2.at[rowv.at[jnp.maximum(t - 1, 0)]],
                        dbufs[b2], gsems[b2]).wait()
                    pltpu.async_copy(dbufs[b2],
                                     acc.at[colv.at[jnp.maximum(t - 1, 0)]],
                                     ssems[b2], add=True)
            return 0
        lax.fori_loop(0, (hpr + 1 + 2) // 3, _steps, 0)

        for b in range(3):
            pltpu.make_async_copy(dbufs[b], acc.at[colv.at[hpr - 3 + b]],
                                  ssems[b]).wait()
    plsc.subcore_barrier()

    pltpu.sync_copy(acc.at[pl.ds(s * 640, 640)],
                    g_o.at[pl.ds(c * NP + s * 640, 640)])


def _a0_body(x_ref, af_ref, wt_ref, dis_ref, o_ref):
    xb = x_ref[...]
    h0 = jnp.zeros((NBLK, EMB), F32)
    for f in range(9):
        oh = (xb[:, f][:, None]
              == lax.broadcasted_iota(I32, (1, 64), 1)).astype(F32)
        h0 = h0 + jnp.dot(oh, af_ref[pl.ds(f * 64, 64), :],
                          preferred_element_type=F32)
    hl = jnp.maximum(jnp.dot(h0, wt_ref[...], preferred_element_type=F32), 0.0)
    hs = hl * dis_ref[...]
    o_ref[0] = hs[:, :128]
    o_ref[1] = hs[:, 128:]


def _atom_layer0(x_p, atom_flat, w0t, dis):
    return pl.pallas_call(
        _a0_body,
        grid=(NGRID,),
        in_specs=[
            pl.BlockSpec((NBLK, 9), lambda i: (i, 0)),
            pl.BlockSpec((576, EMB), lambda i: (0, 0)),
            pl.BlockSpec((EMB, EMB), lambda i: (0, 0)),
            pl.BlockSpec((NBLK, 1), lambda i: (i, 0)),
        ],
        out_specs=pl.BlockSpec((2, NBLK, 128), lambda i: (0, i, 0)),
        out_shape=jax.ShapeDtypeStruct((2, NP, 128), F32),
    )(x_p, atom_flat, w0t, dis)


def _lt_phase1(i, g_ref, t_ref, bf_ref, hs_ref, dis1_ref, sdeg_ref,
               dinv_ref, root_ref, pre_sc, sacc):
    G = jnp.concatenate([g_ref[0], g_ref[1]], axis=1)
    HS = jnp.concatenate([hs_ref[0], hs_ref[1]], axis=1)
    Tb = t_ref[0] + t_ref[1]
    hl = HS * sdeg_ref[...]
    pre = (dis1_ref[...]
           * (G + jnp.dot(Tb, bf_ref[...], preferred_element_type=F32))
           + (hl + root_ref[...]) * dinv_ref[...])
    pre_sc[pl.ds(i * NBLK, NBLK), :] = pre
    gid = i * NBLK + lax.broadcasted_iota(I32, (NBLK, 1), 0)
    pm = jnp.where(gid < N, pre, 0.0)

    @pl.when(i == 0)
    def _():
        sacc[...] = jnp.zeros((8, EMB), F32)

    sacc[0:1, :] = sacc[0:1, :] + jnp.sum(pm, axis=0, keepdims=True)
    sacc[1:2, :] = sacc[1:2, :] + jnp.sum(pm * pm, axis=0, keepdims=True)


def _bn_from(sacc):
    mean = sacc[0:1, :] * (1.0 / N)
    var = sacc[1:2, :] * (1.0 / N) - mean * mean
    return mean, lax.rsqrt(var + 1e-5)


def _lt_body(g_ref, t_ref, bf_ref, hs_ref, dis1_ref, sdeg_ref, dinv_ref,
             root_ref, gm_ref, bt_ref, wt_ref, dis2_ref, o_ref,
             pre_sc, sacc):
    i = pl.program_id(0)

    @pl.when(i < NGRID)
    def _():
        _lt_phase1(i, g_ref, t_ref, bf_ref, hs_ref, dis1_ref, sdeg_ref,
                   dinv_ref, root_ref, pre_sc, sacc)

    @pl.when(i >= NGRID)
    def _():
        ii = i - NGRID
        pre = pre_sc[pl.ds(ii * NBLK, NBLK), :]
        mean, inv = _bn_from(sacc)
        h = (pre - mean) * inv * gm_ref[...] + bt_ref[...]
        h = jnp.maximum(h, 0.0)
        hl = jnp.maximum(jnp.dot(h, wt_ref[...], preferred_element_type=F32),
                         0.0)
        hs = hl * dis2_ref[...]
        o_ref[0] = hs[:, :128]
        o_ref[1] = hs[:, 128:]


def _c1(i):
    return (0, jnp.minimum(i, NGRID - 1), 0)


def _r1(i):
    return (jnp.minimum(i, NGRID - 1), 0)


_P1_SPECS = [
    pl.BlockSpec((2, NBLK, 128), _c1),
    pl.BlockSpec((2, NBLK, 48), _c1),
    pl.BlockSpec((48, EMB), lambda i: (0, 0)),
    pl.BlockSpec((2, NBLK, 128), _c1),
    pl.BlockSpec((NBLK, 1), _r1),
    pl.BlockSpec((NBLK, 1), _r1),
    pl.BlockSpec((NBLK, 1), _r1),
    pl.BlockSpec((1, EMB), lambda i: (0, 0)),
]


def _layer_tc(g3, t3, bondflat, hs3, dis, sdeg, dinv, root_l,
              gamma_l, beta_l, wt):
    return pl.pallas_call(
        _lt_body,
        grid=(2 * NGRID,),
        in_specs=_P1_SPECS + [
            pl.BlockSpec((1, EMB), lambda i: (0, 0)),
            pl.BlockSpec((1, EMB), lambda i: (0, 0)),
            pl.BlockSpec((EMB, EMB), lambda i: (0, 0)),
            pl.BlockSpec((NBLK, 1), lambda i: (jnp.maximum(i - NGRID, 0), 0)),
        ],
        out_specs=pl.BlockSpec((2, NBLK, 128),
                               lambda i: (0, jnp.maximum(i - NGRID, 0), 0)),
        out_shape=jax.ShapeDtypeStruct((2, NP, 128), F32),
        scratch_shapes=[pltpu.VMEM((NP, EMB), F32), pltpu.VMEM((8, EMB), F32)],
    )(g3, t3, bondflat, hs3, dis, sdeg, dinv, root_l, gamma_l, beta_l,
      wt, dis)


FBLK = 1000


def _ltf_body(g_ref, t_ref, bf_ref, hs_ref, dis1_ref, sdeg_ref, dinv_ref,
              root_ref, gm_ref, bt_ref, o_ref, pre_sc, sacc):
    i = pl.program_id(0)

    @pl.when(i < NGRID)
    def _():
        _lt_phase1(i, g_ref, t_ref, bf_ref, hs_ref, dis1_ref, sdeg_ref,
                   dinv_ref, root_ref, pre_sc, sacc)

    @pl.when(i >= NGRID)
    def _():
        ii = i - NGRID
        pre = pre_sc[pl.ds(ii * FBLK, FBLK), :]
        mean, inv = _bn_from(sacc)
        o_ref[...] = (pre - mean) * inv * gm_ref[...] + bt_ref[...]


def _layer_tc_final(g3, t3, bondflat, hs3, dis, sdeg, dinv, root_l,
                    gamma_l, beta_l):
    return pl.pallas_call(
        _ltf_body,
        grid=(2 * NGRID,),
        in_specs=_P1_SPECS + [
            pl.BlockSpec((1, EMB), lambda i: (0, 0)),
            pl.BlockSpec((1, EMB), lambda i: (0, 0)),
        ],
        out_specs=pl.BlockSpec((FBLK, EMB),
                               lambda i: (jnp.maximum(i - NGRID, 0), 0)),
        out_shape=jax.ShapeDtypeStruct((N, EMB), F32),
        scratch_shapes=[pltpu.VMEM((NP, EMB), F32), pltpu.VMEM((8, EMB), F32)],
    )(g3, t3, bondflat, hs3, dis, sdeg, dinv, root_l, gamma_l, beta_l)


def kernel(x, edge_index, edge_attr, atom_tab, W, root, bond, gamma, beta):
    row = edge_index[0].astype(I32)
    col = edge_index[1].astype(I32)
    pad_e = EP - E
    pad_ids = (N + (jnp.arange(pad_e, dtype=I32) % (NP - N))).astype(I32)
    row2d = jnp.concatenate([row, pad_ids]).reshape(EROWS, ECH)
    col2d = jnp.concatenate([col, pad_ids]).reshape(EROWS, ECH)
    eap = jnp.concatenate(
        [edge_attr.astype(I32), jnp.zeros((pad_e, 3), I32)], axis=0)
    ea0 = eap[:, 0].reshape(EROWS, ECH)
    ea1 = eap[:, 1].reshape(EROWS, ECH)
    ea2 = eap[:, 2].reshape(EROWS, ECH)
    x_p = jnp.concatenate(
        [x.astype(I32), jnp.zeros((NP - N, x.shape[1]), I32)], axis=0)

    atom_flat = atom_tab.reshape(576, EMB)
    wts = [W[l].T for l in range(3)]
    bfs = [bond[l].reshape(48, EMB) for l in range(3)]

    deg2 = _sc_deg(row2d).reshape(2, NP // 128, 128)
    dis, sdeg, dinv = _deg_norm(deg2)
    dis = dis.reshape(NP, 1)
    sdeg = sdeg.reshape(NP, 1)
    dinv = dinv.reshape(NP, 1)
    t_flat = _sc_tbins(row2d, col2d, ea0, ea1, ea2, dis.reshape(NP))
    t3 = t_flat.reshape(2, NP, 48)

    hs3 = _atom_layer0(x_p, atom_flat, wts[0], dis)
    out = None
    for l in range(3):
        g2 = _sc_spmm(hs3.reshape(2 * NP, 128),
                      row2d.reshape(SROWS, SCH), col2d.reshape(SROWS, SCH))
        g3 = g2.reshape(2, NP, 128)
        if l < 2:
            hs3 = _layer_tc(g3, t3, bfs[l], hs3, dis, sdeg, dinv,
                            root[l][None, :], gamma[l][None, :],
                            beta[l][None, :], wts[l + 1])
        else:
            out = _layer_tc_final(g3, t3, bfs[l], hs3, dis, sdeg, dinv,
                                  root[l][None, :], gamma[l][None, :],
                                  beta[l][None, :])
    return out
